# baseline (speedup 1.0000x reference)
import functools

import jax
import jax.numpy as jnp
from jax import lax
from jax.experimental import pallas as pl
from jax.experimental.pallas import tpu as pltpu

N = 10000
E = 320000
D = 128
H = 128
G = 256
GF = 4
NC = 2
K = 4
EPS = 1e-5

_HI = lax.Precision.HIGHEST
_INTERPRET = False


def _lrelu(x):
    return jnp.where(x >= 0, x, 0.01 * x)


def _pc(body, out_shape, grid=None, in_specs=None, out_specs=None, scratch_shapes=None):
    kw = {}
    if grid is not None:
        kw["grid"] = grid
    if in_specs is not None:
        kw["in_specs"] = in_specs
    if out_specs is not None:
        kw["out_specs"] = out_specs
    if scratch_shapes is not None:
        kw["scratch_shapes"] = scratch_shapes
    return pl.pallas_call(body, out_shape=out_shape, interpret=_INTERPRET, **kw)


_RB_N = 2000


def _segsum_body(y_ref, bc_ref, s_ref, c_ref, accs, accc):
    i = pl.program_id(0)
    iota_col = lax.broadcasted_iota(jnp.int32, (G, 1), 0)
    MT = (iota_col == bc_ref[0]).astype(jnp.float32)
    ps = jnp.dot(MT, y_ref[...], precision=_HI, preferred_element_type=jnp.float32)
    pc_ = jnp.sum(MT, axis=1, keepdims=True)

    @pl.when(i == 0)
    def _():
        accs[...] = jnp.zeros_like(accs)
        accc[...] = jnp.zeros_like(accc)

    accs[...] += ps
    accc[...] += pc_

    @pl.when(i == pl.num_programs(0) - 1)
    def _():
        s_ref[...] = accs[...]
        c_ref[...] = accc[...]


def _segsum(y, bc3):
    c = y.shape[1]
    return _pc(
        _segsum_body,
        (jax.ShapeDtypeStruct((G, c), jnp.float32),
         jax.ShapeDtypeStruct((G, 1), jnp.float32)),
        grid=(N // _RB_N,),
        in_specs=[pl.BlockSpec((_RB_N, c), lambda i: (i, 0)),
                  pl.BlockSpec((1, 1, _RB_N), lambda i: (i, 0, 0))],
        out_specs=(pl.BlockSpec((G, c), lambda i: (0, 0)),
                   pl.BlockSpec((G, 1), lambda i: (0, 0))),
        scratch_shapes=[pltpu.VMEM((G, c), jnp.float32),
                        pltpu.VMEM((G, 1), jnp.float32)],
    )(y, bc3)


def _gn_mid_body(x_ref, br_ref, bc_ref, s_ref, c_ref, ms_ref,
                 o_ref, v_ref, accv):
    i = pl.program_id(0)
    cnt = jnp.maximum(c_ref[...], 1.0)
    mean = s_ref[...] / cnt
    iota_row = lax.broadcasted_iota(jnp.int32, (1, G), 1)
    M = (br_ref[...] == iota_row).astype(jnp.float32)
    meanb = jnp.dot(M, mean, precision=_HI, preferred_element_type=jnp.float32)
    out0 = x_ref[...] - meanb * ms_ref[...]
    o_ref[...] = out0
    iota_col = lax.broadcasted_iota(jnp.int32, (G, 1), 0)
    MT = (iota_col == bc_ref[0]).astype(jnp.float32)
    pv = jnp.dot(MT, out0 * out0, precision=_HI,
                 preferred_element_type=jnp.float32)

    @pl.when(i == 0)
    def _():
        accv[...] = jnp.zeros_like(accv)

    accv[...] += pv

    @pl.when(i == pl.num_programs(0) - 1)
    def _():
        v_ref[...] = accv[...]


def _gn_fin_body(o0_ref, br_ref, v_ref, c_ref, w_ref, b_ref, o_ref):
    cnt = jnp.maximum(c_ref[...], 1.0)
    var = v_ref[...] / cnt
    iota_row = lax.broadcasted_iota(jnp.int32, (1, G), 1)
    M = (br_ref[...] == iota_row).astype(jnp.float32)
    varb = jnp.dot(M, var, precision=_HI, preferred_element_type=jnp.float32)
    o_ref[...] = w_ref[...] * o0_ref[...] / jnp.sqrt(varb + EPS) + b_ref[...]


def _graph_norm(x, br, bc3, p):
    sums, cnt = _segsum(x, bc3)
    out0, vsums = _pc(
        _gn_mid_body,
        (jax.ShapeDtypeStruct((N, D), jnp.float32),
         jax.ShapeDtypeStruct((G, D), jnp.float32)),
        grid=(N // _RB_N,),
        in_specs=[pl.BlockSpec((_RB_N, D), lambda i: (i, 0)),
                  pl.BlockSpec((_RB_N, 1), lambda i: (i, 0)),
                  pl.BlockSpec((1, 1, _RB_N), lambda i: (i, 0, 0)),
                  pl.BlockSpec((G, D), lambda i: (0, 0)),
                  pl.BlockSpec((G, 1), lambda i: (0, 0)),
                  pl.BlockSpec((1, D), lambda i: (0, 0))],
        out_specs=(pl.BlockSpec((_RB_N, D), lambda i: (i, 0)),
                   pl.BlockSpec((G, D), lambda i: (0, 0))),
        scratch_shapes=[pltpu.VMEM((G, D), jnp.float32)],
    )(x, br, bc3, sums, cnt, p["ms"].reshape(1, D))
    return _pc(
        _gn_fin_body,
        jax.ShapeDtypeStruct((N, D), jnp.float32),
        grid=(N // _RB_N,),
        in_specs=[pl.BlockSpec((_RB_N, D), lambda i: (i, 0)),
                  pl.BlockSpec((_RB_N, 1), lambda i: (i, 0)),
                  pl.BlockSpec((G, D), lambda i: (0, 0)),
                  pl.BlockSpec((G, 1), lambda i: (0, 0)),
                  pl.BlockSpec((1, D), lambda i: (0, 0)),
                  pl.BlockSpec((1, D), lambda i: (0, 0))],
        out_specs=pl.BlockSpec((_RB_N, D), lambda i: (i, 0)),
    )(out0, br, vsums, cnt, p["w"].reshape(1, D), p["b"].reshape(1, D))


def _mm_body(x_ref, w_ref, b_ref, o_ref):
    o_ref[...] = jnp.dot(x_ref[...], w_ref[...],
                         preferred_element_type=jnp.float32) + b_ref[...]


def _node_mm(x, wt, b):
    n, _ = x.shape
    co = wt.shape[1]
    return _pc(_mm_body, jax.ShapeDtypeStruct((n, co), jnp.float32))(
        x, wt, b.reshape(1, co))


_RB_E = 2048


def _mlp1_body(ne, e_ref, w_ref, b_ref, o_ref, so_ref, acc):
    i = pl.program_id(0)
    out = _lrelu(jnp.dot(e_ref[...], w_ref[...],
                         preferred_element_type=jnp.float32) + b_ref[...])
    o_ref[...] = out

    @pl.when(i == 0)
    def _():
        acc[...] = jnp.zeros_like(acc)

    rb = out.shape[0]
    gid = i * rb + lax.broadcasted_iota(jnp.int32, (rb, 1), 0)
    outm = jnp.where(gid < ne, out, 0.0)
    acc[0:1, :] += jnp.sum(outm, axis=0, keepdims=True)
    acc[1:2, :] += jnp.sum(outm * outm, axis=0, keepdims=True)

    @pl.when(i == pl.num_programs(0) - 1)
    def _():
        so_ref[...] = acc[...]


def _mlp_pass1(ecat, wt, b, ne):
    ep = ecat.shape[0]
    nblk = ep // _RB_E
    body = functools.partial(_mlp1_body, ne)
    return _pc(
        body,
        (jax.ShapeDtypeStruct((ep, H), jnp.float32),
         jax.ShapeDtypeStruct((2, H), jnp.float32)),
        grid=(nblk,),
        in_specs=[
            pl.BlockSpec((_RB_E, 2 * H), lambda i: (i, 0)),
            pl.BlockSpec((2 * H, H), lambda i: (0, 0)),
            pl.BlockSpec((1, H), lambda i: (0, 0)),
        ],
        out_specs=(pl.BlockSpec((_RB_E, H), lambda i: (i, 0)),
                   pl.BlockSpec((2, H), lambda i: (0, 0))),
        scratch_shapes=[pltpu.VMEM((2, H), jnp.float32)],
    )(ecat, wt, b.reshape(1, H))


def _mlp_pass_body(ne, e_ref, st_ref, w_ref, bcur_ref, gprev_ref, beprev_ref,
                   o_ref, so_ref, acc):
    i = pl.program_id(0)
    st = st_ref[...]
    mean = st[0:1, :] / ne
    var = st[1:2, :] / ne - mean * mean
    en = (e_ref[...] - mean) / jnp.sqrt(var + EPS) * gprev_ref[...] + beprev_ref[...]
    out = _lrelu(jnp.dot(en, w_ref[...], preferred_element_type=jnp.float32)
                 + bcur_ref[...])
    o_ref[...] = out

    @pl.when(i == 0)
    def _():
        acc[...] = jnp.zeros_like(acc)

    rb = out.shape[0]
    gid = i * rb + lax.broadcasted_iota(jnp.int32, (rb, 1), 0)
    outm = jnp.where(gid < ne, out, 0.0)
    acc[0:1, :] += jnp.sum(outm, axis=0, keepdims=True)
    acc[1:2, :] += jnp.sum(outm * outm, axis=0, keepdims=True)

    @pl.when(i == pl.num_programs(0) - 1)
    def _():
        so_ref[...] = acc[...]


def _mlp_pass(e, stats, wt, b_cur, g_prev, be_prev, ne):
    ep = e.shape[0]
    nblk = ep // _RB_E
    body = functools.partial(_mlp_pass_body, float(ne))
    return _pc(
        body,
        (jax.ShapeDtypeStruct((ep, H), jnp.float32),
         jax.ShapeDtypeStruct((2, H), jnp.float32)),
        grid=(nblk,),
        in_specs=[
            pl.BlockSpec((_RB_E, H), lambda i: (i, 0)),
            pl.BlockSpec((2, H), lambda i: (0, 0)),
            pl.BlockSpec((H, H), lambda i: (0, 0)),
            pl.BlockSpec((1, H), lambda i: (0, 0)),
            pl.BlockSpec((1, H), lambda i: (0, 0)),
            pl.BlockSpec((1, H), lambda i: (0, 0)),
        ],
        out_specs=(pl.BlockSpec((_RB_E, H), lambda i: (i, 0)),
                   pl.BlockSpec((2, H), lambda i: (0, 0))),
        scratch_shapes=[pltpu.VMEM((2, H), jnp.float32)],
    )(e, stats, wt, b_cur.reshape(1, H), g_prev.reshape(1, H), be_prev.reshape(1, H))


def _bn_sc(sc, g, be):
    m = jnp.mean(sc, axis=0, keepdims=True)
    d = sc - m
    v = jnp.mean(d * d, axis=0, keepdims=True)
    return d / jnp.sqrt(v + EPS) * g + be


def _econv_fin_body(ne, acc_ref, cnt_ref, st3_ref, g3_ref, be3_ref,
                    sc_ref, gsc_ref, besc_ref, o_ref):
    st = st3_ref[...]
    mean3 = st[0:1, :] / ne
    var3 = st[1:2, :] / ne - mean3 * mean3
    s3 = g3_ref[...] / jnp.sqrt(var3 + EPS)
    c3 = be3_ref[...] - mean3 * s3
    accs = jnp.sum(acc_ref[...], axis=0)
    cnt = jnp.sum(cnt_ref[...], axis=0)
    agg = (accs * s3 + c3 * cnt) / jnp.maximum(cnt, 1.0)
    o_ref[...] = agg + _bn_sc(sc_ref[...], gsc_ref[...], besc_ref[...])


def _econv_finalize(acc, cnt, st3, q3, sc_pre, qsc, ne):
    body = functools.partial(_econv_fin_body, float(ne))
    return _pc(body, jax.ShapeDtypeStruct((N, H), jnp.float32))(
        acc, cnt, st3, q3["g"].reshape(1, H), q3["be"].reshape(1, H),
        sc_pre, qsc["g"].reshape(1, H), qsc["be"].reshape(1, H))


def _knn_fin_body(ne, e_ref, st3_ref, g3_ref, be3_ref, sc_ref, gsc_ref,
                  besc_ref, o_ref):
    st = st3_ref[...]
    mean3 = st[0:1, :] / ne
    var3 = st[1:2, :] / ne - mean3 * mean3
    s3 = g3_ref[...] / jnp.sqrt(var3 + EPS)
    c3 = be3_ref[...] - mean3 * s3
    e = e_ref[...]
    mn = (e[:, 0:H] + e[:, H:2 * H] + e[:, 2 * H:3 * H] + e[:, 3 * H:4 * H]) * 0.25
    agg = mn * s3 + c3
    o_ref[...] = agg + _bn_sc(sc_ref[...], gsc_ref[...], besc_ref[...])


def _knn_finalize(e3r, st3, q3, sc_pre, qsc, ne):
    body = functools.partial(_knn_fin_body, float(ne))
    return _pc(body, jax.ShapeDtypeStruct((N, H), jnp.float32))(
        e3r, st3, q3["g"].reshape(1, H), q3["be"].reshape(1, H),
        sc_pre, qsc["g"].reshape(1, H), qsc["be"].reshape(1, H))


_RB_TK = 400
_CT_TK = 512
_NP_TK = 10240
_BIGI = 1 << 30


def _topk_body(x_ref, d2c_ref, d2r_ref, br_blk_ref, bc_ref, idx_ref):
    i = pl.program_id(0)
    R = _RB_TK
    CT = _CT_TK
    xr = x_ref[pl.ds(i * R, R), :]
    d2r_blk = d2c_ref[pl.ds(i * R, R), :]
    br = br_blk_ref[...]
    b_lo = br[0, 0]
    b_hi = br[R - 1, 0]
    iota_n = lax.broadcasted_iota(jnp.int32, (1, _NP_TK), 1)
    bc_all = bc_ref[...]
    cmin = jnp.min(jnp.where(bc_all[0:1, :] == b_lo, iota_n, _BIGI))
    cmax = jnp.max(jnp.where(bc_all[0:1, :] == b_hi, iota_n, -1)) + 1
    t0 = cmin // CT
    t1 = (cmax + CT - 1) // CT
    row_ids = i * R + lax.broadcasted_iota(jnp.int32, (R, 1), 0)
    inf = jnp.float32(jnp.inf)

    def _sel4(nv, ni):
        bvs = []
        bis = []
        for _ in range(K):
            m = jnp.min(nv, axis=1, keepdims=True)
            mi = jnp.min(jnp.where(nv == m, ni, _BIGI), axis=1, keepdims=True)
            nv = jnp.where(ni == mi, inf, nv)
            bvs.append(m)
            bis.append(mi)
        return jnp.concatenate(bvs, axis=1), jnp.concatenate(bis, axis=1)

    def tile_step(t, carry):
        bv, bi = carry
        c0 = pl.multiple_of(t * CT, CT)
        xc = x_ref[pl.ds(c0, CT), :]
        qk = lax.dot_general(xr, xc, (((1,), (1,)), ((), ())),
                             preferred_element_type=jnp.float32)
        d2c_row = d2r_ref[:, pl.ds(c0, CT)]
        dist = d2r_blk + d2c_row - 2.0 * qk
        bct = bc_ref[:, pl.ds(c0, CT)]
        col_ids = c0 + lax.broadcasted_iota(jnp.int32, (1, CT), 1)
        bad = (bct != br) | (col_ids == row_ids)
        cand = jnp.where(bad, inf, dist)
        tv = []
        ti = []
        for _ in range(K):
            tm = jnp.min(cand, axis=1, keepdims=True)
            tix = jnp.min(jnp.where(cand == tm, col_ids, _BIGI),
                          axis=1, keepdims=True)
            cand = jnp.where(col_ids == tix, inf, cand)
            tv.append(tm)
            ti.append(tix)
        nv = jnp.concatenate([bv] + tv, axis=1)
        ni = jnp.concatenate([bi] + ti, axis=1)
        return _sel4(nv, ni)

    bv0 = jnp.full((R, K), inf, jnp.float32)
    bi0 = jnp.full((R, K), _BIGI, jnp.int32)
    bv, bi = lax.fori_loop(t0, t1, tile_step, (bv0, bi0))

    s0 = t0 * CT
    s1 = jnp.minimum(t1 * CT, N)
    kk = lax.broadcasted_iota(jnp.int32, (1, K), 1)
    ids_out = jnp.where(kk < s0, kk, s1 + kk - s0)
    nv = jnp.concatenate([bv, jnp.full((R, K), inf, jnp.float32)], axis=1)
    ni = jnp.concatenate([bi, jnp.broadcast_to(ids_out, (R, K))], axis=1)
    bv, bi = _sel4(nv, ni)
    idx_ref[...] = bi


def _knn_topk(x_pad, d2c, d2r_pad, br, bc_pad):
    return _pc(
        _topk_body,
        jax.ShapeDtypeStruct((N, K), jnp.int32),
        grid=(N // _RB_TK,),
        in_specs=[
            pl.BlockSpec((_NP_TK, D), lambda i: (0, 0)),
            pl.BlockSpec((N, 1), lambda i: (0, 0)),
            pl.BlockSpec((1, _NP_TK), lambda i: (0, 0)),
            pl.BlockSpec((_RB_TK, 1), lambda i: (i, 0)),
            pl.BlockSpec((1, _NP_TK), lambda i: (0, 0)),
        ],
        out_specs=pl.BlockSpec((_RB_TK, K), lambda i: (i, 0)),
    )(x_pad, d2c, d2r_pad, br, bc_pad)


def _bn_rows(x, g, be):
    m = jnp.mean(x, axis=0, keepdims=True)
    d = x - m
    v = jnp.mean(d * d, axis=0, keepdims=True)
    return d / jnp.sqrt(v + EPS) * g + be


def _head_body(p1_ref, p2_ref, p3_ref, gi_ref, cnt_ref,
               g0a_ref, b0a_ref, g0b_ref, b0b_ref, g0c_ref, b0c_ref,
               g0g_ref, b0g_ref,
               w1a_ref, w1b_ref, w1c_ref, w1g_ref, b1_ref, g1_ref, be1_ref,
               w2_ref, b2_ref, g2_ref, be2_ref, wo_ref, bo_ref, o_ref):
    cnt = jnp.maximum(cnt_ref[...], 1.0)
    p1 = p1_ref[...] / cnt
    p2 = p2_ref[...] / cnt
    p3 = p3_ref[...] / cnt
    gi = gi_ref[...]
    p1 = _bn_rows(p1, g0a_ref[...], b0a_ref[...])
    p2 = _bn_rows(p2, g0b_ref[...], b0b_ref[...])
    p3 = _bn_rows(p3, g0c_ref[...], b0c_ref[...])
    gi = _bn_rows(gi, g0g_ref[...], b0g_ref[...])
    h = (jnp.dot(p1, w1a_ref[...], preferred_element_type=jnp.float32)
         + jnp.dot(p2, w1b_ref[...], preferred_element_type=jnp.float32)
         + jnp.dot(p3, w1c_ref[...], preferred_element_type=jnp.float32)
         + jnp.dot(gi, w1g_ref[...], preferred_element_type=jnp.float32)
         + b1_ref[...])
    h = _bn_rows(_lrelu(h), g1_ref[...], be1_ref[...])
    h = _lrelu(jnp.dot(h, w2_ref[...], preferred_element_type=jnp.float32)
               + b2_ref[...])
    h = _bn_rows(h, g2_ref[...], be2_ref[...])
    lo = jnp.dot(h, wo_ref[...], preferred_element_type=jnp.float32) + bo_ref[...]
    m = jnp.max(lo, axis=1, keepdims=True)
    ex = jnp.exp(lo - m)
    o_ref[...] = ex / jnp.sum(ex, axis=1, keepdims=True)


def _head(c1, c2, c3, gi, bc3, hp):
    p1s, cnt = _segsum(c1, bc3)
    p2s, _ = _segsum(c2, bc3)
    p3s, _ = _segsum(c3, bc3)
    w1t = hp["d1_W"].T
    args = [
        p1s, p2s, p3s, gi, cnt,
        hp["bn0_g"][0:H].reshape(1, H), hp["bn0_b"][0:H].reshape(1, H),
        hp["bn0_g"][H:2 * H].reshape(1, H), hp["bn0_b"][H:2 * H].reshape(1, H),
        hp["bn0_g"][2 * H:3 * H].reshape(1, H), hp["bn0_b"][2 * H:3 * H].reshape(1, H),
        hp["bn0_g"][3 * H:].reshape(1, GF), hp["bn0_b"][3 * H:].reshape(1, GF),
        w1t[0:H], w1t[H:2 * H], w1t[2 * H:3 * H], w1t[3 * H:],
        hp["d1_b"].reshape(1, H), hp["bn1_g"].reshape(1, H), hp["bn1_b"].reshape(1, H),
        hp["d2_W"].T, hp["d2_b"].reshape(1, H),
        hp["bn2_g"].reshape(1, H), hp["bn2_b"].reshape(1, H),
        hp["out_W"].T, hp["out_b"].reshape(1, NC),
    ]
    return _pc(_head_body, jax.ShapeDtypeStruct((G, NC), jnp.float32))(*args)


_CE = 128
_NW = 32
_HALF = 5120
_ACC_R = 6144
_LDUMP = 6136
_DUMP = 10232


def _epad(ne):
    return ((ne + _NW * _CE - 1) // (_NW * _CE)) * (_NW * _CE)


def _edge_gather(x, srcp, dstp, ep):
    from jax.experimental.pallas import tpu_sc as plsc
    nch = ep // (_NW * _CE)
    mesh = plsc.VectorSubcoreMesh(core_axis_name="c", subcore_axis_name="s")

    @functools.partial(
        pl.kernel, mesh=mesh,
        out_type=jax.ShapeDtypeStruct((ep, 2 * H), jnp.float32),
        scratch_types=[
            pltpu.VMEM((_CE,), jnp.int32),
            pltpu.VMEM((_CE,), jnp.int32),
            pltpu.VMEM((_CE, H), jnp.float32),
            pltpu.VMEM((_CE, H), jnp.float32),
            pltpu.VMEM((_CE, 2 * H), jnp.float32),
            pltpu.SemaphoreType.DMA,
            pltpu.SemaphoreType.DMA,
        ])
    def k(x_hbm, src_hbm, dst_hbm, out_hbm, si, di, ri, rj, ro, sem1, sem2):
        wid = lax.axis_index("s") * 2 + lax.axis_index("c")
        base0 = wid * (ep // _NW)

        def chunk(j, carry):
            base = base0 + j * _CE
            pltpu.sync_copy(dst_hbm.at[pl.ds(base, _CE)], di)
            pltpu.sync_copy(src_hbm.at[pl.ds(base, _CE)], si)
            cp1 = pltpu.async_copy(x_hbm.at[di], ri, sem1)
            cp2 = pltpu.async_copy(x_hbm.at[si], rj, sem2)
            cp1.wait()
            cp2.wait()

            def row(r, c2):
                for g in range(H // 16):
                    a = ri[r, pl.ds(g * 16, 16)]
                    b = rj[r, pl.ds(g * 16, 16)]
                    ro[r, pl.ds(g * 16, 16)] = a
                    ro[r, pl.ds(H + g * 16, 16)] = b - a
                return c2

            lax.fori_loop(0, _CE, row, 0)
            pltpu.sync_copy(ro, out_hbm.at[pl.ds(base, _CE)])
            return carry

        lax.fori_loop(0, nch, chunk, 0)

    return k(x, srcp, dstp)


def _edge_scatter(e3, dstp, ne):
    from jax.experimental.pallas import tpu_sc as plsc
    ep = e3.shape[0]
    nch = ep // (16 * _CE)
    mesh = plsc.VectorSubcoreMesh(core_axis_name="c", subcore_axis_name="s")
    rpt = _ACC_R // 16

    @functools.partial(
        pl.kernel, mesh=mesh,
        out_type=[jax.ShapeDtypeStruct((2, _ACC_R, H), jnp.float32),
                  jax.ShapeDtypeStruct((2, _ACC_R, 16), jnp.float32)],
        scratch_types=[
            pltpu.VMEM((_CE,), jnp.int32),
            pltpu.VMEM((1, _CE), jnp.int32),
            pltpu.VMEM((_CE, H), jnp.float32),
            pltpu.VMEM((_CE, 16), jnp.float32),
            pltpu.VMEM((_CE, 16), jnp.float32),
            pltpu.VMEM_SHARED((_ACC_R, H), jnp.float32),
            pltpu.VMEM_SHARED((_ACC_R, 16), jnp.float32),
            pltpu.SemaphoreType.DMA,
        ])
    def k(e_hbm, dst_hbm, acc_hbm, cnt_hbm, di, dl2, rows, ones_v, z16, acc_s,
          cnt_s, sem):
        cid = lax.axis_index("c")
        sid = lax.axis_index("s")
        lo = cid * _HALF

        def zrow(r, c2):
            for g in range(H // 16):
                rows[r, pl.ds(g * 16, 16)] = jnp.zeros((16,), jnp.float32)
            ones_v[r, pl.ds(0, 16)] = jnp.full((16,), 1.0, jnp.float32)
            z16[r, pl.ds(0, 16)] = jnp.zeros((16,), jnp.float32)
            return c2

        lax.fori_loop(0, _CE, zrow, 0)
        for z in range(rpt // _CE):
            pltpu.sync_copy(rows, acc_s.at[pl.ds(sid * rpt + z * _CE, _CE)])
            pltpu.sync_copy(z16, cnt_s.at[pl.ds(sid * rpt + z * _CE, _CE)])
        plsc.subcore_barrier()

        base0 = sid * (ep // 16)

        def chunk(j, carry):
            base = base0 + j * _CE
            pltpu.sync_copy(dst_hbm.at[pl.ds(base, _CE)], di)
            pltpu.async_copy(e_hbm.at[pl.ds(base, _CE)], rows, sem).wait()
            for g in range(_CE // 16):
                dv = di[pl.ds(g * 16, 16)]
                dl = dv - lo
                ok = (dl >= 0) & (dl < _HALF)
                dl2[0, pl.ds(g * 16, 16)] = jnp.where(ok, dl, _LDUMP)
            idxrow = dl2.at[0]
            pltpu.sync_copy(rows, acc_s.at[idxrow], add=True)
            pltpu.sync_copy(ones_v, cnt_s.at[idxrow], add=True)
            return carry

        lax.fori_loop(0, nch, chunk, 0)
        plsc.subcore_barrier()
        for z in range(rpt // _CE):
            r0 = sid * rpt + z * _CE
            pltpu.sync_copy(acc_s.at[pl.ds(r0, _CE)], rows)
            pltpu.sync_copy(rows, acc_hbm.at[cid, pl.ds(r0, _CE)])
            pltpu.sync_copy(cnt_s.at[pl.ds(r0, _CE)], ones_v)
            pltpu.sync_copy(ones_v, cnt_hbm.at[cid, pl.ds(r0, _CE)])

    acc, cnt = k(e3, dstp)
    accf = jnp.concatenate([acc[0, :_HALF], acc[1, :_HALF]], axis=0)[:N]
    cntf = jnp.concatenate([cnt[0, :_HALF], cnt[1, :_HALF]], axis=0)[:N]
    return accf.reshape(1, N, H), cntf[:, 0:1].reshape(1, N, 1)


def _edge_scatter_jnp(e3, dstp, ne):
    acc = jax.ops.segment_sum(e3[:ne], dstp[:ne], num_segments=N)
    cnt = jax.ops.segment_sum(jnp.ones((ne,), jnp.float32), dstp[:ne],
                              num_segments=N)
    return acc.reshape(1, N, H), cnt.reshape(1, N, 1)


def _edge_mlp(ecat, q, ne):
    e1, st1 = _mlp_pass1(ecat, q["m1"]["W"].T, q["m1"]["b"], ne)
    e2, st2 = _mlp_pass(e1, st1, q["m2"]["W"].T, q["m2"]["b"],
                        q["m1"]["g"], q["m1"]["be"], ne)
    e3, st3 = _mlp_pass(e2, st2, q["m3"]["W"].T, q["m3"]["b"],
                        q["m2"]["g"], q["m2"]["be"], ne)
    return e3, st3


def _edge_conv(xin, src, dst, q):
    sc_pre = _node_mm(xin, q["sc"]["W"].T, q["sc"]["b"])
    ep = _epad(E)
    pad = ep - E
    zpad = jnp.zeros((pad,), jnp.int32)
    srcp = jnp.concatenate([src, zpad])
    dstg = jnp.concatenate([dst, zpad])
    dsts = jnp.concatenate([dst, jnp.full((pad,), _DUMP, jnp.int32)])
    ecat = _edge_gather(xin, srcp, dstg, ep)
    e3, st3 = _edge_mlp(ecat, q, E)
    acc, cnt = _edge_scatter_jnp(e3, dsts, E)
    return _econv_finalize(acc, cnt, st3, q["m3"], sc_pre, q["sc"], E)


def _knn_conv(xin, br, bc, dstk, q):
    sc_pre = _node_mm(xin, q["sc"]["W"].T, q["sc"]["b"])
    d2 = jnp.sum(xin * xin, axis=1)
    x_pad = jnp.concatenate(
        [xin, jnp.zeros((_NP_TK - N, D), jnp.float32)], axis=0)
    d2r_pad = jnp.concatenate(
        [d2, jnp.zeros((_NP_TK - N,), jnp.float32)]).reshape(1, _NP_TK)
    bc_pad = jnp.concatenate(
        [bc[0], jnp.full((_NP_TK - N,), -1, jnp.int32)]).reshape(1, _NP_TK)
    idx = _knn_topk(x_pad, d2.reshape(N, 1), d2r_pad, br, bc_pad)
    nk = N * K
    ep = _epad(nk)
    zpad = jnp.zeros((ep - nk,), jnp.int32)
    srck = jnp.concatenate([idx.reshape(-1), zpad])
    dstkp = jnp.concatenate([dstk, zpad])
    ecat = _edge_gather(xin, srck, dstkp, ep)
    e3, st3 = _edge_mlp(ecat, q, nk)
    return _knn_finalize(e3[:nk].reshape(N, K * H), st3, q["m3"],
                         sc_pre, q["sc"], nk)


def kernel(x, edge_index, graph_input, batch, params):
    br = batch.reshape(N, 1)
    bc = batch.reshape(1, N)
    src = edge_index[0]
    dst = edge_index[1]
    dstk = jnp.repeat(jnp.arange(N, dtype=jnp.int32), K)

    bc3 = batch.reshape(N // _RB_N, 1, _RB_N)
    xg = _graph_norm(x, br, bc3, params["gn"])
    c1 = _edge_conv(xg, src, dst, params["conv1"])
    c2 = _knn_conv(c1, br, bc, dstk, params["conv2"])
    c3 = _knn_conv(c2, br, bc, dstk, params["conv3"])
    return _head(c1, c2, c3, graph_input, bc3, params["head"])


# speedup vs baseline: 1.1401x; 1.1401x over previous
import functools

import jax
import jax.numpy as jnp
from jax import lax
from jax.experimental import pallas as pl
from jax.experimental.pallas import tpu as pltpu

N = 10000
E = 320000
D = 128
H = 128
G = 256
GF = 4
NC = 2
K = 4
EPS = 1e-5

_HI = lax.Precision.HIGHEST
_INTERPRET = False


def _lrelu(x):
    return jnp.where(x >= 0, x, 0.01 * x)


def _pc(body, out_shape, grid=None, in_specs=None, out_specs=None, scratch_shapes=None):
    kw = {}
    if grid is not None:
        kw["grid"] = grid
    if in_specs is not None:
        kw["in_specs"] = in_specs
    if out_specs is not None:
        kw["out_specs"] = out_specs
    if scratch_shapes is not None:
        kw["scratch_shapes"] = scratch_shapes
    return pl.pallas_call(body, out_shape=out_shape, interpret=_INTERPRET, **kw)


_RB_N = 2000


def _segsum_body(y_ref, bc_ref, s_ref, c_ref, accs, accc):
    i = pl.program_id(0)
    iota_col = lax.broadcasted_iota(jnp.int32, (G, 1), 0)
    MT = (iota_col == bc_ref[0]).astype(jnp.float32)
    ps = jnp.dot(MT, y_ref[...], precision=_HI, preferred_element_type=jnp.float32)
    pc_ = jnp.sum(MT, axis=1, keepdims=True)

    @pl.when(i == 0)
    def _():
        accs[...] = jnp.zeros_like(accs)
        accc[...] = jnp.zeros_like(accc)

    accs[...] += ps
    accc[...] += pc_

    @pl.when(i == pl.num_programs(0) - 1)
    def _():
        s_ref[...] = accs[...]
        c_ref[...] = accc[...]


def _segsum(y, bc3):
    c = y.shape[1]
    return _pc(
        _segsum_body,
        (jax.ShapeDtypeStruct((G, c), jnp.float32),
         jax.ShapeDtypeStruct((G, 1), jnp.float32)),
        grid=(N // _RB_N,),
        in_specs=[pl.BlockSpec((_RB_N, c), lambda i: (i, 0)),
                  pl.BlockSpec((1, 1, _RB_N), lambda i: (i, 0, 0))],
        out_specs=(pl.BlockSpec((G, c), lambda i: (0, 0)),
                   pl.BlockSpec((G, 1), lambda i: (0, 0))),
        scratch_shapes=[pltpu.VMEM((G, c), jnp.float32),
                        pltpu.VMEM((G, 1), jnp.float32)],
    )(y, bc3)


def _gn_mid_body(x_ref, br_ref, bc_ref, s_ref, c_ref, ms_ref,
                 o_ref, v_ref, accv):
    i = pl.program_id(0)
    cnt = jnp.maximum(c_ref[...], 1.0)
    mean = s_ref[...] / cnt
    iota_row = lax.broadcasted_iota(jnp.int32, (1, G), 1)
    M = (br_ref[...] == iota_row).astype(jnp.float32)
    meanb = jnp.dot(M, mean, precision=_HI, preferred_element_type=jnp.float32)
    out0 = x_ref[...] - meanb * ms_ref[...]
    o_ref[...] = out0
    iota_col = lax.broadcasted_iota(jnp.int32, (G, 1), 0)
    MT = (iota_col == bc_ref[0]).astype(jnp.float32)
    pv = jnp.dot(MT, out0 * out0, precision=_HI,
                 preferred_element_type=jnp.float32)

    @pl.when(i == 0)
    def _():
        accv[...] = jnp.zeros_like(accv)

    accv[...] += pv

    @pl.when(i == pl.num_programs(0) - 1)
    def _():
        v_ref[...] = accv[...]


def _gn_fin_body(o0_ref, br_ref, v_ref, c_ref, w_ref, b_ref, o_ref):
    cnt = jnp.maximum(c_ref[...], 1.0)
    var = v_ref[...] / cnt
    iota_row = lax.broadcasted_iota(jnp.int32, (1, G), 1)
    M = (br_ref[...] == iota_row).astype(jnp.float32)
    varb = jnp.dot(M, var, precision=_HI, preferred_element_type=jnp.float32)
    o_ref[...] = w_ref[...] * o0_ref[...] / jnp.sqrt(varb + EPS) + b_ref[...]


def _graph_norm(x, br, bc3, p):
    sums, cnt = _segsum(x, bc3)
    out0, vsums = _pc(
        _gn_mid_body,
        (jax.ShapeDtypeStruct((N, D), jnp.float32),
         jax.ShapeDtypeStruct((G, D), jnp.float32)),
        grid=(N // _RB_N,),
        in_specs=[pl.BlockSpec((_RB_N, D), lambda i: (i, 0)),
                  pl.BlockSpec((_RB_N, 1), lambda i: (i, 0)),
                  pl.BlockSpec((1, 1, _RB_N), lambda i: (i, 0, 0)),
                  pl.BlockSpec((G, D), lambda i: (0, 0)),
                  pl.BlockSpec((G, 1), lambda i: (0, 0)),
                  pl.BlockSpec((1, D), lambda i: (0, 0))],
        out_specs=(pl.BlockSpec((_RB_N, D), lambda i: (i, 0)),
                   pl.BlockSpec((G, D), lambda i: (0, 0))),
        scratch_shapes=[pltpu.VMEM((G, D), jnp.float32)],
    )(x, br, bc3, sums, cnt, p["ms"].reshape(1, D))
    return _pc(
        _gn_fin_body,
        jax.ShapeDtypeStruct((N, D), jnp.float32),
        grid=(N // _RB_N,),
        in_specs=[pl.BlockSpec((_RB_N, D), lambda i: (i, 0)),
                  pl.BlockSpec((_RB_N, 1), lambda i: (i, 0)),
                  pl.BlockSpec((G, D), lambda i: (0, 0)),
                  pl.BlockSpec((G, 1), lambda i: (0, 0)),
                  pl.BlockSpec((1, D), lambda i: (0, 0)),
                  pl.BlockSpec((1, D), lambda i: (0, 0))],
        out_specs=pl.BlockSpec((_RB_N, D), lambda i: (i, 0)),
    )(out0, br, vsums, cnt, p["w"].reshape(1, D), p["b"].reshape(1, D))


def _mm_body(x_ref, w_ref, b_ref, o_ref):
    o_ref[...] = jnp.dot(x_ref[...], w_ref[...],
                         preferred_element_type=jnp.float32) + b_ref[...]


def _node_mm(x, wt, b):
    n, _ = x.shape
    co = wt.shape[1]
    return _pc(_mm_body, jax.ShapeDtypeStruct((n, co), jnp.float32))(
        x, wt, b.reshape(1, co))


_RB_E = 2048


def _mlp1_body(ne, xi_ref, xj_ref, w_ref, b_ref, o_ref, so_ref, acc):
    i = pl.program_id(0)
    xi = xi_ref[...]
    cat = jnp.concatenate([xi, xj_ref[...] - xi], axis=1)
    out = _lrelu(jnp.dot(cat, w_ref[...],
                         preferred_element_type=jnp.float32) + b_ref[...])
    o_ref[...] = out

    @pl.when(i == 0)
    def _():
        acc[...] = jnp.zeros_like(acc)

    rb = out.shape[0]
    gid = i * rb + lax.broadcasted_iota(jnp.int32, (rb, 1), 0)
    outm = jnp.where(gid < ne, out, 0.0)
    acc[0:1, :] += jnp.sum(outm, axis=0, keepdims=True)
    acc[1:2, :] += jnp.sum(outm * outm, axis=0, keepdims=True)

    @pl.when(i == pl.num_programs(0) - 1)
    def _():
        so_ref[...] = acc[...]


def _mlp_pass1(xi, xj, wt, b, ne):
    ep = xi.shape[0]
    nblk = ep // _RB_E
    body = functools.partial(_mlp1_body, ne)
    return _pc(
        body,
        (jax.ShapeDtypeStruct((ep, H), jnp.float32),
         jax.ShapeDtypeStruct((2, H), jnp.float32)),
        grid=(nblk,),
        in_specs=[
            pl.BlockSpec((_RB_E, H), lambda i: (i, 0)),
            pl.BlockSpec((_RB_E, H), lambda i: (i, 0)),
            pl.BlockSpec((2 * H, H), lambda i: (0, 0)),
            pl.BlockSpec((1, H), lambda i: (0, 0)),
        ],
        out_specs=(pl.BlockSpec((_RB_E, H), lambda i: (i, 0)),
                   pl.BlockSpec((2, H), lambda i: (0, 0))),
        scratch_shapes=[pltpu.VMEM((2, H), jnp.float32)],
    )(xi, xj, wt, b.reshape(1, H))


def _mlp_pass_body(ne, e_ref, st_ref, w_ref, bcur_ref, gprev_ref, beprev_ref,
                   o_ref, so_ref, acc):
    i = pl.program_id(0)
    st = st_ref[...]
    mean = st[0:1, :] / ne
    var = st[1:2, :] / ne - mean * mean
    en = (e_ref[...] - mean) / jnp.sqrt(var + EPS) * gprev_ref[...] + beprev_ref[...]
    out = _lrelu(jnp.dot(en, w_ref[...], preferred_element_type=jnp.float32)
                 + bcur_ref[...])
    o_ref[...] = out

    @pl.when(i == 0)
    def _():
        acc[...] = jnp.zeros_like(acc)

    rb = out.shape[0]
    gid = i * rb + lax.broadcasted_iota(jnp.int32, (rb, 1), 0)
    outm = jnp.where(gid < ne, out, 0.0)
    acc[0:1, :] += jnp.sum(outm, axis=0, keepdims=True)
    acc[1:2, :] += jnp.sum(outm * outm, axis=0, keepdims=True)

    @pl.when(i == pl.num_programs(0) - 1)
    def _():
        so_ref[...] = acc[...]


def _mlp_pass(e, stats, wt, b_cur, g_prev, be_prev, ne):
    ep = e.shape[0]
    nblk = ep // _RB_E
    body = functools.partial(_mlp_pass_body, float(ne))
    return _pc(
        body,
        (jax.ShapeDtypeStruct((ep, H), jnp.float32),
         jax.ShapeDtypeStruct((2, H), jnp.float32)),
        grid=(nblk,),
        in_specs=[
            pl.BlockSpec((_RB_E, H), lambda i: (i, 0)),
            pl.BlockSpec((2, H), lambda i: (0, 0)),
            pl.BlockSpec((H, H), lambda i: (0, 0)),
            pl.BlockSpec((1, H), lambda i: (0, 0)),
            pl.BlockSpec((1, H), lambda i: (0, 0)),
            pl.BlockSpec((1, H), lambda i: (0, 0)),
        ],
        out_specs=(pl.BlockSpec((_RB_E, H), lambda i: (i, 0)),
                   pl.BlockSpec((2, H), lambda i: (0, 0))),
        scratch_shapes=[pltpu.VMEM((2, H), jnp.float32)],
    )(e, stats, wt, b_cur.reshape(1, H), g_prev.reshape(1, H), be_prev.reshape(1, H))


def _bn_sc(sc, g, be):
    m = jnp.mean(sc, axis=0, keepdims=True)
    d = sc - m
    v = jnp.mean(d * d, axis=0, keepdims=True)
    return d / jnp.sqrt(v + EPS) * g + be


def _econv_fin_body(ne, acc_ref, cnt_ref, st3_ref, g3_ref, be3_ref,
                    sc_ref, gsc_ref, besc_ref, o_ref):
    st = st3_ref[...]
    mean3 = st[0:1, :] / ne
    var3 = st[1:2, :] / ne - mean3 * mean3
    s3 = g3_ref[...] / jnp.sqrt(var3 + EPS)
    c3 = be3_ref[...] - mean3 * s3
    accs = jnp.sum(acc_ref[...], axis=0)
    cnt = jnp.sum(cnt_ref[...], axis=0)
    agg = (accs * s3 + c3 * cnt) / jnp.maximum(cnt, 1.0)
    o_ref[...] = agg + _bn_sc(sc_ref[...], gsc_ref[...], besc_ref[...])


def _econv_finalize(acc, cnt, st3, q3, sc_pre, qsc, ne):
    body = functools.partial(_econv_fin_body, float(ne))
    return _pc(body, jax.ShapeDtypeStruct((N, H), jnp.float32))(
        acc, cnt, st3, q3["g"].reshape(1, H), q3["be"].reshape(1, H),
        sc_pre, qsc["g"].reshape(1, H), qsc["be"].reshape(1, H))


def _knn_fin_body(ne, e_ref, st3_ref, g3_ref, be3_ref, sc_ref, gsc_ref,
                  besc_ref, o_ref):
    st = st3_ref[...]
    mean3 = st[0:1, :] / ne
    var3 = st[1:2, :] / ne - mean3 * mean3
    s3 = g3_ref[...] / jnp.sqrt(var3 + EPS)
    c3 = be3_ref[...] - mean3 * s3
    e = e_ref[...]
    mn = (e[:, 0:H] + e[:, H:2 * H] + e[:, 2 * H:3 * H] + e[:, 3 * H:4 * H]) * 0.25
    agg = mn * s3 + c3
    o_ref[...] = agg + _bn_sc(sc_ref[...], gsc_ref[...], besc_ref[...])


def _knn_finalize(e3r, st3, q3, sc_pre, qsc, ne):
    body = functools.partial(_knn_fin_body, float(ne))
    return _pc(body, jax.ShapeDtypeStruct((N, H), jnp.float32))(
        e3r, st3, q3["g"].reshape(1, H), q3["be"].reshape(1, H),
        sc_pre, qsc["g"].reshape(1, H), qsc["be"].reshape(1, H))


_RB_TK = 400
_CT_TK = 512
_NP_TK = 10240
_BIGI = 1 << 30


def _topk_body(x_ref, d2c_ref, d2r_ref, br_blk_ref, bc_ref, idx_ref):
    i = pl.program_id(0)
    R = _RB_TK
    CT = _CT_TK
    xr = x_ref[pl.ds(i * R, R), :]
    d2r_blk = d2c_ref[pl.ds(i * R, R), :]
    br = br_blk_ref[...]
    b_lo = br[0, 0]
    b_hi = br[R - 1, 0]
    iota_n = lax.broadcasted_iota(jnp.int32, (1, _NP_TK), 1)
    bc_all = bc_ref[...]
    cmin = jnp.min(jnp.where(bc_all[0:1, :] == b_lo, iota_n, _BIGI))
    cmax = jnp.max(jnp.where(bc_all[0:1, :] == b_hi, iota_n, -1)) + 1
    t0 = cmin // CT
    t1 = (cmax + CT - 1) // CT
    row_ids = i * R + lax.broadcasted_iota(jnp.int32, (R, 1), 0)
    inf = jnp.float32(jnp.inf)

    def _sel4(nv, ni):
        bvs = []
        bis = []
        for _ in range(K):
            m = jnp.min(nv, axis=1, keepdims=True)
            mi = jnp.min(jnp.where(nv == m, ni, _BIGI), axis=1, keepdims=True)
            nv = jnp.where(ni == mi, inf, nv)
            bvs.append(m)
            bis.append(mi)
        return jnp.concatenate(bvs, axis=1), jnp.concatenate(bis, axis=1)

    def tile_step(t, carry):
        bv, bi = carry
        c0 = pl.multiple_of(t * CT, CT)
        xc = x_ref[pl.ds(c0, CT), :]
        qk = lax.dot_general(xr, xc, (((1,), (1,)), ((), ())),
                             preferred_element_type=jnp.float32)
        d2c_row = d2r_ref[:, pl.ds(c0, CT)]
        dist = d2r_blk + d2c_row - 2.0 * qk
        bct = bc_ref[:, pl.ds(c0, CT)]
        col_ids = c0 + lax.broadcasted_iota(jnp.int32, (1, CT), 1)
        bad = (bct != br) | (col_ids == row_ids)
        cand = jnp.where(bad, inf, dist)
        tv = []
        ti = []
        for _ in range(K):
            tm = jnp.min(cand, axis=1, keepdims=True)
            tix = jnp.min(jnp.where(cand == tm, col_ids, _BIGI),
                          axis=1, keepdims=True)
            cand = jnp.where(col_ids == tix, inf, cand)
            tv.append(tm)
            ti.append(tix)
        nv = jnp.concatenate([bv] + tv, axis=1)
        ni = jnp.concatenate([bi] + ti, axis=1)
        return _sel4(nv, ni)

    bv0 = jnp.full((R, K), inf, jnp.float32)
    bi0 = jnp.full((R, K), _BIGI, jnp.int32)
    bv, bi = lax.fori_loop(t0, t1, tile_step, (bv0, bi0))

    s0 = t0 * CT
    s1 = jnp.minimum(t1 * CT, N)
    kk = lax.broadcasted_iota(jnp.int32, (1, K), 1)
    ids_out = jnp.where(kk < s0, kk, s1 + kk - s0)
    nv = jnp.concatenate([bv, jnp.full((R, K), inf, jnp.float32)], axis=1)
    ni = jnp.concatenate([bi, jnp.broadcast_to(ids_out, (R, K))], axis=1)
    bv, bi = _sel4(nv, ni)
    idx_ref[...] = bi


def _knn_topk(x_pad, d2c, d2r_pad, br, bc_pad):
    return _pc(
        _topk_body,
        jax.ShapeDtypeStruct((N, K), jnp.int32),
        grid=(N // _RB_TK,),
        in_specs=[
            pl.BlockSpec((_NP_TK, D), lambda i: (0, 0)),
            pl.BlockSpec((N, 1), lambda i: (0, 0)),
            pl.BlockSpec((1, _NP_TK), lambda i: (0, 0)),
            pl.BlockSpec((_RB_TK, 1), lambda i: (i, 0)),
            pl.BlockSpec((1, _NP_TK), lambda i: (0, 0)),
        ],
        out_specs=pl.BlockSpec((_RB_TK, K), lambda i: (i, 0)),
    )(x_pad, d2c, d2r_pad, br, bc_pad)


def _bn_rows(x, g, be):
    m = jnp.mean(x, axis=0, keepdims=True)
    d = x - m
    v = jnp.mean(d * d, axis=0, keepdims=True)
    return d / jnp.sqrt(v + EPS) * g + be


def _head_body(p1_ref, p2_ref, p3_ref, gi_ref, cnt_ref,
               g0a_ref, b0a_ref, g0b_ref, b0b_ref, g0c_ref, b0c_ref,
               g0g_ref, b0g_ref,
               w1a_ref, w1b_ref, w1c_ref, w1g_ref, b1_ref, g1_ref, be1_ref,
               w2_ref, b2_ref, g2_ref, be2_ref, wo_ref, bo_ref, o_ref):
    cnt = jnp.maximum(cnt_ref[...], 1.0)
    p1 = p1_ref[...] / cnt
    p2 = p2_ref[...] / cnt
    p3 = p3_ref[...] / cnt
    gi = gi_ref[...]
    p1 = _bn_rows(p1, g0a_ref[...], b0a_ref[...])
    p2 = _bn_rows(p2, g0b_ref[...], b0b_ref[...])
    p3 = _bn_rows(p3, g0c_ref[...], b0c_ref[...])
    gi = _bn_rows(gi, g0g_ref[...], b0g_ref[...])
    h = (jnp.dot(p1, w1a_ref[...], preferred_element_type=jnp.float32)
         + jnp.dot(p2, w1b_ref[...], preferred_element_type=jnp.float32)
         + jnp.dot(p3, w1c_ref[...], preferred_element_type=jnp.float32)
         + jnp.dot(gi, w1g_ref[...], preferred_element_type=jnp.float32)
         + b1_ref[...])
    h = _bn_rows(_lrelu(h), g1_ref[...], be1_ref[...])
    h = _lrelu(jnp.dot(h, w2_ref[...], preferred_element_type=jnp.float32)
               + b2_ref[...])
    h = _bn_rows(h, g2_ref[...], be2_ref[...])
    lo = jnp.dot(h, wo_ref[...], preferred_element_type=jnp.float32) + bo_ref[...]
    m = jnp.max(lo, axis=1, keepdims=True)
    ex = jnp.exp(lo - m)
    o_ref[...] = ex / jnp.sum(ex, axis=1, keepdims=True)


def _head(c1, c2, c3, gi, bc3, hp):
    p1s, cnt = _segsum(c1, bc3)
    p2s, _ = _segsum(c2, bc3)
    p3s, _ = _segsum(c3, bc3)
    w1t = hp["d1_W"].T
    args = [
        p1s, p2s, p3s, gi, cnt,
        hp["bn0_g"][0:H].reshape(1, H), hp["bn0_b"][0:H].reshape(1, H),
        hp["bn0_g"][H:2 * H].reshape(1, H), hp["bn0_b"][H:2 * H].reshape(1, H),
        hp["bn0_g"][2 * H:3 * H].reshape(1, H), hp["bn0_b"][2 * H:3 * H].reshape(1, H),
        hp["bn0_g"][3 * H:].reshape(1, GF), hp["bn0_b"][3 * H:].reshape(1, GF),
        w1t[0:H], w1t[H:2 * H], w1t[2 * H:3 * H], w1t[3 * H:],
        hp["d1_b"].reshape(1, H), hp["bn1_g"].reshape(1, H), hp["bn1_b"].reshape(1, H),
        hp["d2_W"].T, hp["d2_b"].reshape(1, H),
        hp["bn2_g"].reshape(1, H), hp["bn2_b"].reshape(1, H),
        hp["out_W"].T, hp["out_b"].reshape(1, NC),
    ]
    return _pc(_head_body, jax.ShapeDtypeStruct((G, NC), jnp.float32))(*args)


_CE = 128
_NW = 32
_HALF = 5120
_ACC_R = 6144
_LDUMP = 6136
_DUMP = 10232


def _epad(ne):
    return ((ne + _NW * _CE - 1) // (_NW * _CE)) * (_NW * _CE)


def _edge_gather(x, sd2, ep):
    from jax.experimental.pallas import tpu_sc as plsc
    nch = ep // (_NW * _CE)
    mesh = plsc.VectorSubcoreMesh(core_axis_name="c", subcore_axis_name="s")

    @functools.partial(
        pl.kernel, mesh=mesh,
        out_type=[jax.ShapeDtypeStruct((ep, H), jnp.float32),
                  jax.ShapeDtypeStruct((ep, H), jnp.float32)],
        scratch_types=[
            pltpu.VMEM((2, _CE), jnp.int32),
            pltpu.VMEM((2, _CE), jnp.int32),
            pltpu.VMEM((_CE, H), jnp.float32),
            pltpu.VMEM((_CE, H), jnp.float32),
            pltpu.VMEM((_CE, H), jnp.float32),
            pltpu.VMEM((_CE, H), jnp.float32),
            pltpu.SemaphoreType.DMA,
            pltpu.SemaphoreType.DMA,
            pltpu.SemaphoreType.DMA,
            pltpu.SemaphoreType.DMA,
        ])
    def k(x_hbm, sd_hbm, oi_hbm, oj_hbm,
          ix0, ix1, bi0, bj0, bi1, bj1, sg0, sg1, so0, so1):
        wid = lax.axis_index("s") * 2 + lax.axis_index("c")
        base0 = wid * (ep // _NW)
        ix = (ix0, ix1)
        bi = (bi0, bi1)
        bj = (bj0, bj1)
        sg = (sg0, sg1)
        so = (so0, so1)

        def load_fire(j, s):
            base = base0 + j * _CE
            pltpu.sync_copy(sd_hbm.at[:, pl.ds(base, _CE)], ix[s])
            pltpu.async_copy(x_hbm.at[ix[s].at[0]], bi[s], sg[s])
            pltpu.async_copy(x_hbm.at[ix[s].at[1]], bj[s], sg[s])

        def drain_g(s):
            pltpu.make_async_copy(x_hbm.at[ix[s].at[0]], bi[s], sg[s]).wait()
            pltpu.make_async_copy(x_hbm.at[ix[s].at[1]], bj[s], sg[s]).wait()

        def fire_out(j, s):
            base = base0 + j * _CE
            pltpu.async_copy(bi[s], oi_hbm.at[pl.ds(base, _CE)], so[s])
            pltpu.async_copy(bj[s], oj_hbm.at[pl.ds(base, _CE)], so[s])

        def drain_out(s):
            pltpu.make_async_copy(bi[s], oi_hbm.at[pl.ds(0, _CE)], so[s]).wait()
            pltpu.make_async_copy(bj[s], oj_hbm.at[pl.ds(0, _CE)], so[s]).wait()

        load_fire(0, 0)

        def pair(t2, carry):
            for s in (0, 1):
                j = 2 * t2 + s

                @pl.when(j < nch)
                def _():
                    drain_g(s)

                    @pl.when(j + 1 < nch)
                    def _():
                        @pl.when(j >= 1)
                        def _():
                            drain_out(1 - s)

                        load_fire(j + 1, 1 - s)

                    fire_out(j, s)
            return carry

        lax.fori_loop(0, (nch + 1) // 2, pair, 0)
        drain_out((nch - 1) % 2)
        drain_out(nch % 2)

    return k(x, sd2)


def _edge_scatter(e3, dstp, ne):
    from jax.experimental.pallas import tpu_sc as plsc
    ep = e3.shape[0]
    nch = ep // (16 * _CE)
    mesh = plsc.VectorSubcoreMesh(core_axis_name="c", subcore_axis_name="s")
    rpt = _ACC_R // 16

    @functools.partial(
        pl.kernel, mesh=mesh,
        out_type=[jax.ShapeDtypeStruct((2, _ACC_R, H), jnp.float32),
                  jax.ShapeDtypeStruct((2, _ACC_R, 16), jnp.float32)],
        scratch_types=[
            pltpu.VMEM((_CE,), jnp.int32),
            pltpu.VMEM((1, _CE), jnp.int32),
            pltpu.VMEM((_CE, H), jnp.float32),
            pltpu.VMEM((_CE, 16), jnp.float32),
            pltpu.VMEM((_CE, 16), jnp.float32),
            pltpu.VMEM_SHARED((_ACC_R, H), jnp.float32),
            pltpu.VMEM_SHARED((_ACC_R, 16), jnp.float32),
            pltpu.SemaphoreType.DMA,
        ])
    def k(e_hbm, dst_hbm, acc_hbm, cnt_hbm, di, dl2, rows, ones_v, z16, acc_s,
          cnt_s, sem):
        cid = lax.axis_index("c")
        sid = lax.axis_index("s")
        lo = cid * _HALF

        def zrow(r, c2):
            for g in range(H // 16):
                rows[r, pl.ds(g * 16, 16)] = jnp.zeros((16,), jnp.float32)
            ones_v[r, pl.ds(0, 16)] = jnp.full((16,), 1.0, jnp.float32)
            z16[r, pl.ds(0, 16)] = jnp.zeros((16,), jnp.float32)
            return c2

        lax.fori_loop(0, _CE, zrow, 0)
        for z in range(rpt // _CE):
            pltpu.sync_copy(rows, acc_s.at[pl.ds(sid * rpt + z * _CE, _CE)])
            pltpu.sync_copy(z16, cnt_s.at[pl.ds(sid * rpt + z * _CE, _CE)])
        plsc.subcore_barrier()

        base0 = sid * (ep // 16)

        def chunk(j, carry):
            base = base0 + j * _CE
            pltpu.sync_copy(dst_hbm.at[pl.ds(base, _CE)], di)
            pltpu.async_copy(e_hbm.at[pl.ds(base, _CE)], rows, sem).wait()
            for g in range(_CE // 16):
                dv = di[pl.ds(g * 16, 16)]
                dl = dv - lo
                ok = (dl >= 0) & (dl < _HALF)
                dl2[0, pl.ds(g * 16, 16)] = jnp.where(ok, dl, _LDUMP)
            idxrow = dl2.at[0]
            pltpu.sync_copy(rows, acc_s.at[idxrow], add=True)
            pltpu.sync_copy(ones_v, cnt_s.at[idxrow], add=True)
            return carry

        lax.fori_loop(0, nch, chunk, 0)
        plsc.subcore_barrier()
        for z in range(rpt // _CE):
            r0 = sid * rpt + z * _CE
            pltpu.sync_copy(acc_s.at[pl.ds(r0, _CE)], rows)
            pltpu.sync_copy(rows, acc_hbm.at[cid, pl.ds(r0, _CE)])
            pltpu.sync_copy(cnt_s.at[pl.ds(r0, _CE)], ones_v)
            pltpu.sync_copy(ones_v, cnt_hbm.at[cid, pl.ds(r0, _CE)])

    acc, cnt = k(e3, dstp)
    accf = jnp.concatenate([acc[0, :_HALF], acc[1, :_HALF]], axis=0)[:N]
    cntf = jnp.concatenate([cnt[0, :_HALF], cnt[1, :_HALF]], axis=0)[:N]
    return accf.reshape(1, N, H), cntf[:, 0:1].reshape(1, N, 1)


def _edge_scatter_jnp(e3, dstp, ne):
    acc = jax.ops.segment_sum(e3[:ne], dstp[:ne], num_segments=N)
    cnt = jax.ops.segment_sum(jnp.ones((ne,), jnp.float32), dstp[:ne],
                              num_segments=N)
    return acc.reshape(1, N, H), cnt.reshape(1, N, 1)


def _edge_mlp(xi, xj, q, ne):
    e1, st1 = _mlp_pass1(xi, xj, q["m1"]["W"].T, q["m1"]["b"], ne)
    e2, st2 = _mlp_pass(e1, st1, q["m2"]["W"].T, q["m2"]["b"],
                        q["m1"]["g"], q["m1"]["be"], ne)
    e3, st3 = _mlp_pass(e2, st2, q["m3"]["W"].T, q["m3"]["b"],
                        q["m2"]["g"], q["m2"]["be"], ne)
    return e3, st3


def _edge_conv(xin, src, dst, q):
    sc_pre = _node_mm(xin, q["sc"]["W"].T, q["sc"]["b"])
    ep = _epad(E)
    pad = ep - E
    zpad = jnp.zeros((pad,), jnp.int32)
    srcp = jnp.concatenate([src, zpad])
    dstg = jnp.concatenate([dst, zpad])
    dsts = jnp.concatenate([dst, jnp.full((pad,), _DUMP, jnp.int32)])
    xi, xj = _edge_gather(xin, jnp.stack([dstg, srcp]), ep)
    e3, st3 = _edge_mlp(xi, xj, q, E)
    acc, cnt = _edge_scatter_jnp(e3, dsts, E)
    return _econv_finalize(acc, cnt, st3, q["m3"], sc_pre, q["sc"], E)


def _knn_conv(xin, br, bc, dstk, q):
    sc_pre = _node_mm(xin, q["sc"]["W"].T, q["sc"]["b"])
    d2 = jnp.sum(xin * xin, axis=1)
    x_pad = jnp.concatenate(
        [xin, jnp.zeros((_NP_TK - N, D), jnp.float32)], axis=0)
    d2r_pad = jnp.concatenate(
        [d2, jnp.zeros((_NP_TK - N,), jnp.float32)]).reshape(1, _NP_TK)
    bc_pad = jnp.concatenate(
        [bc[0], jnp.full((_NP_TK - N,), -1, jnp.int32)]).reshape(1, _NP_TK)
    idx = _knn_topk(x_pad, d2.reshape(N, 1), d2r_pad, br, bc_pad)
    nk = N * K
    ep = _epad(nk)
    zpad = jnp.zeros((ep - nk,), jnp.int32)
    srck = jnp.concatenate([idx.reshape(-1), zpad])
    dstkp = jnp.concatenate([dstk, zpad])
    xi, xj = _edge_gather(xin, jnp.stack([dstkp, srck]), ep)
    e3, st3 = _edge_mlp(xi, xj, q, nk)
    return _knn_finalize(e3[:nk].reshape(N, K * H), st3, q["m3"],
                         sc_pre, q["sc"], nk)


def kernel(x, edge_index, graph_input, batch, params):
    br = batch.reshape(N, 1)
    bc = batch.reshape(1, N)
    src = edge_index[0]
    dst = edge_index[1]
    dstk = jnp.repeat(jnp.arange(N, dtype=jnp.int32), K)

    bc3 = batch.reshape(N // _RB_N, 1, _RB_N)
    xg = _graph_norm(x, br, bc3, params["gn"])
    c1 = _edge_conv(xg, src, dst, params["conv1"])
    c2 = _knn_conv(c1, br, bc, dstk, params["conv2"])
    c3 = _knn_conv(c2, br, bc, dstk, params["conv3"])
    return _head(c1, c2, c3, graph_input, bc3, params["head"])


# speedup vs baseline: 1.2153x; 1.0659x over previous
import functools

import jax
import jax.numpy as jnp
from jax import lax
from jax.experimental import pallas as pl
from jax.experimental.pallas import tpu as pltpu

N = 10000
E = 320000
D = 128
H = 128
G = 256
GF = 4
NC = 2
K = 4
EPS = 1e-5

_HI = lax.Precision.HIGHEST
_INTERPRET = False


def _lrelu(x):
    return jnp.where(x >= 0, x, 0.01 * x)


def _pc(body, out_shape, grid=None, in_specs=None, out_specs=None, scratch_shapes=None):
    kw = {}
    if grid is not None:
        kw["grid"] = grid
    if in_specs is not None:
        kw["in_specs"] = in_specs
    if out_specs is not None:
        kw["out_specs"] = out_specs
    if scratch_shapes is not None:
        kw["scratch_shapes"] = scratch_shapes
    return pl.pallas_call(body, out_shape=out_shape, interpret=_INTERPRET, **kw)


_RB_N = 2000


def _segsum_body(y_ref, bc_ref, s_ref, c_ref, accs, accc):
    i = pl.program_id(0)
    iota_col = lax.broadcasted_iota(jnp.int32, (G, 1), 0)
    MT = (iota_col == bc_ref[0]).astype(jnp.float32)
    ps = jnp.dot(MT, y_ref[...], precision=_HI, preferred_element_type=jnp.float32)
    pc_ = jnp.sum(MT, axis=1, keepdims=True)

    @pl.when(i == 0)
    def _():
        accs[...] = jnp.zeros_like(accs)
        accc[...] = jnp.zeros_like(accc)

    accs[...] += ps
    accc[...] += pc_

    @pl.when(i == pl.num_programs(0) - 1)
    def _():
        s_ref[...] = accs[...]
        c_ref[...] = accc[...]


def _segsum(y, bc3):
    c = y.shape[1]
    return _pc(
        _segsum_body,
        (jax.ShapeDtypeStruct((G, c), jnp.float32),
         jax.ShapeDtypeStruct((G, 1), jnp.float32)),
        grid=(N // _RB_N,),
        in_specs=[pl.BlockSpec((_RB_N, c), lambda i: (i, 0)),
                  pl.BlockSpec((1, 1, _RB_N), lambda i: (i, 0, 0))],
        out_specs=(pl.BlockSpec((G, c), lambda i: (0, 0)),
                   pl.BlockSpec((G, 1), lambda i: (0, 0))),
        scratch_shapes=[pltpu.VMEM((G, c), jnp.float32),
                        pltpu.VMEM((G, 1), jnp.float32)],
    )(y, bc3)


def _gn_mid_body(x_ref, br_ref, bc_ref, s_ref, c_ref, ms_ref,
                 o_ref, v_ref, accv):
    i = pl.program_id(0)
    cnt = jnp.maximum(c_ref[...], 1.0)
    mean = s_ref[...] / cnt
    iota_row = lax.broadcasted_iota(jnp.int32, (1, G), 1)
    M = (br_ref[...] == iota_row).astype(jnp.float32)
    meanb = jnp.dot(M, mean, precision=_HI, preferred_element_type=jnp.float32)
    out0 = x_ref[...] - meanb * ms_ref[...]
    o_ref[...] = out0
    iota_col = lax.broadcasted_iota(jnp.int32, (G, 1), 0)
    MT = (iota_col == bc_ref[0]).astype(jnp.float32)
    pv = jnp.dot(MT, out0 * out0, precision=_HI,
                 preferred_element_type=jnp.float32)

    @pl.when(i == 0)
    def _():
        accv[...] = jnp.zeros_like(accv)

    accv[...] += pv

    @pl.when(i == pl.num_programs(0) - 1)
    def _():
        v_ref[...] = accv[...]


def _gn_fin_body(o0_ref, br_ref, v_ref, c_ref, w_ref, b_ref, o_ref):
    cnt = jnp.maximum(c_ref[...], 1.0)
    var = v_ref[...] / cnt
    iota_row = lax.broadcasted_iota(jnp.int32, (1, G), 1)
    M = (br_ref[...] == iota_row).astype(jnp.float32)
    varb = jnp.dot(M, var, precision=_HI, preferred_element_type=jnp.float32)
    o_ref[...] = w_ref[...] * o0_ref[...] / jnp.sqrt(varb + EPS) + b_ref[...]


def _graph_norm(x, br, bc3, p):
    sums, cnt = _segsum(x, bc3)
    out0, vsums = _pc(
        _gn_mid_body,
        (jax.ShapeDtypeStruct((N, D), jnp.float32),
         jax.ShapeDtypeStruct((G, D), jnp.float32)),
        grid=(N // _RB_N,),
        in_specs=[pl.BlockSpec((_RB_N, D), lambda i: (i, 0)),
                  pl.BlockSpec((_RB_N, 1), lambda i: (i, 0)),
                  pl.BlockSpec((1, 1, _RB_N), lambda i: (i, 0, 0)),
                  pl.BlockSpec((G, D), lambda i: (0, 0)),
                  pl.BlockSpec((G, 1), lambda i: (0, 0)),
                  pl.BlockSpec((1, D), lambda i: (0, 0))],
        out_specs=(pl.BlockSpec((_RB_N, D), lambda i: (i, 0)),
                   pl.BlockSpec((G, D), lambda i: (0, 0))),
        scratch_shapes=[pltpu.VMEM((G, D), jnp.float32)],
    )(x, br, bc3, sums, cnt, p["ms"].reshape(1, D))
    return _pc(
        _gn_fin_body,
        jax.ShapeDtypeStruct((N, D), jnp.float32),
        grid=(N // _RB_N,),
        in_specs=[pl.BlockSpec((_RB_N, D), lambda i: (i, 0)),
                  pl.BlockSpec((_RB_N, 1), lambda i: (i, 0)),
                  pl.BlockSpec((G, D), lambda i: (0, 0)),
                  pl.BlockSpec((G, 1), lambda i: (0, 0)),
                  pl.BlockSpec((1, D), lambda i: (0, 0)),
                  pl.BlockSpec((1, D), lambda i: (0, 0))],
        out_specs=pl.BlockSpec((_RB_N, D), lambda i: (i, 0)),
    )(out0, br, vsums, cnt, p["w"].reshape(1, D), p["b"].reshape(1, D))


def _mm_body(x_ref, w_ref, b_ref, o_ref):
    o_ref[...] = jnp.dot(x_ref[...], w_ref[...],
                         preferred_element_type=jnp.float32) + b_ref[...]


def _node_mm(x, wt, b):
    n, _ = x.shape
    co = wt.shape[1]
    return _pc(_mm_body, jax.ShapeDtypeStruct((n, co), jnp.float32))(
        x, wt, b.reshape(1, co))


_RB_E = 4096


def _mlp1_body(ne, xi_ref, xj_ref, w_ref, b_ref, o_ref, so_ref, acc):
    i = pl.program_id(0)
    xi = xi_ref[...]
    cat = jnp.concatenate([xi, xj_ref[...] - xi], axis=1)
    out = _lrelu(jnp.dot(cat, w_ref[...],
                         preferred_element_type=jnp.float32) + b_ref[...])
    o_ref[...] = out

    @pl.when(i == 0)
    def _():
        acc[...] = jnp.zeros_like(acc)

    rb = out.shape[0]
    gid = i * rb + lax.broadcasted_iota(jnp.int32, (rb, 1), 0)
    outm = jnp.where(gid < ne, out, 0.0)
    acc[0:1, :] += jnp.sum(outm, axis=0, keepdims=True)
    acc[1:2, :] += jnp.sum(outm * outm, axis=0, keepdims=True)

    @pl.when(i == pl.num_programs(0) - 1)
    def _():
        so_ref[...] = acc[...]


def _mlp_pass1(xi, xj, wt, b, ne):
    ep = xi.shape[0]
    nblk = ep // _RB_E
    body = functools.partial(_mlp1_body, ne)
    return _pc(
        body,
        (jax.ShapeDtypeStruct((ep, H), jnp.float32),
         jax.ShapeDtypeStruct((2, H), jnp.float32)),
        grid=(nblk,),
        in_specs=[
            pl.BlockSpec((_RB_E, H), lambda i: (i, 0)),
            pl.BlockSpec((_RB_E, H), lambda i: (i, 0)),
            pl.BlockSpec((2 * H, H), lambda i: (0, 0)),
            pl.BlockSpec((1, H), lambda i: (0, 0)),
        ],
        out_specs=(pl.BlockSpec((_RB_E, H), lambda i: (i, 0)),
                   pl.BlockSpec((2, H), lambda i: (0, 0))),
        scratch_shapes=[pltpu.VMEM((2, H), jnp.float32)],
    )(xi, xj, wt, b.reshape(1, H))


def _mlp_pass_body(ne, e_ref, st_ref, w_ref, bcur_ref, gprev_ref, beprev_ref,
                   o_ref, so_ref, acc):
    i = pl.program_id(0)
    st = st_ref[...]
    mean = st[0:1, :] / ne
    var = st[1:2, :] / ne - mean * mean
    en = (e_ref[...] - mean) / jnp.sqrt(var + EPS) * gprev_ref[...] + beprev_ref[...]
    out = _lrelu(jnp.dot(en, w_ref[...], preferred_element_type=jnp.float32)
                 + bcur_ref[...])
    o_ref[...] = out

    @pl.when(i == 0)
    def _():
        acc[...] = jnp.zeros_like(acc)

    rb = out.shape[0]
    gid = i * rb + lax.broadcasted_iota(jnp.int32, (rb, 1), 0)
    outm = jnp.where(gid < ne, out, 0.0)
    acc[0:1, :] += jnp.sum(outm, axis=0, keepdims=True)
    acc[1:2, :] += jnp.sum(outm * outm, axis=0, keepdims=True)

    @pl.when(i == pl.num_programs(0) - 1)
    def _():
        so_ref[...] = acc[...]


def _mlp_pass(e, stats, wt, b_cur, g_prev, be_prev, ne):
    ep = e.shape[0]
    nblk = ep // _RB_E
    body = functools.partial(_mlp_pass_body, float(ne))
    return _pc(
        body,
        (jax.ShapeDtypeStruct((ep, H), jnp.float32),
         jax.ShapeDtypeStruct((2, H), jnp.float32)),
        grid=(nblk,),
        in_specs=[
            pl.BlockSpec((_RB_E, H), lambda i: (i, 0)),
            pl.BlockSpec((2, H), lambda i: (0, 0)),
            pl.BlockSpec((H, H), lambda i: (0, 0)),
            pl.BlockSpec((1, H), lambda i: (0, 0)),
            pl.BlockSpec((1, H), lambda i: (0, 0)),
            pl.BlockSpec((1, H), lambda i: (0, 0)),
        ],
        out_specs=(pl.BlockSpec((_RB_E, H), lambda i: (i, 0)),
                   pl.BlockSpec((2, H), lambda i: (0, 0))),
        scratch_shapes=[pltpu.VMEM((2, H), jnp.float32)],
    )(e, stats, wt, b_cur.reshape(1, H), g_prev.reshape(1, H), be_prev.reshape(1, H))


def _bn_sc(sc, g, be):
    m = jnp.mean(sc, axis=0, keepdims=True)
    d = sc - m
    v = jnp.mean(d * d, axis=0, keepdims=True)
    return d / jnp.sqrt(v + EPS) * g + be


def _econv_fin_body(ne, acc_ref, cnt_ref, st3_ref, g3_ref, be3_ref,
                    sc_ref, gsc_ref, besc_ref, o_ref):
    st = st3_ref[...]
    mean3 = st[0:1, :] / ne
    var3 = st[1:2, :] / ne - mean3 * mean3
    s3 = g3_ref[...] / jnp.sqrt(var3 + EPS)
    c3 = be3_ref[...] - mean3 * s3
    accs = jnp.sum(acc_ref[...], axis=0)
    cnt = jnp.sum(cnt_ref[...], axis=0)
    agg = (accs * s3 + c3 * cnt) / jnp.maximum(cnt, 1.0)
    o_ref[...] = agg + _bn_sc(sc_ref[...], gsc_ref[...], besc_ref[...])


def _econv_finalize(acc, cnt, st3, q3, sc_pre, qsc, ne):
    body = functools.partial(_econv_fin_body, float(ne))
    return _pc(body, jax.ShapeDtypeStruct((N, H), jnp.float32))(
        acc, cnt, st3, q3["g"].reshape(1, H), q3["be"].reshape(1, H),
        sc_pre, qsc["g"].reshape(1, H), qsc["be"].reshape(1, H))


def _knn_fin_body(ne, e_ref, st3_ref, g3_ref, be3_ref, sc_ref, gsc_ref,
                  besc_ref, o_ref):
    st = st3_ref[...]
    mean3 = st[0:1, :] / ne
    var3 = st[1:2, :] / ne - mean3 * mean3
    s3 = g3_ref[...] / jnp.sqrt(var3 + EPS)
    c3 = be3_ref[...] - mean3 * s3
    e = e_ref[...]
    mn = (e[:, 0:H] + e[:, H:2 * H] + e[:, 2 * H:3 * H] + e[:, 3 * H:4 * H]) * 0.25
    agg = mn * s3 + c3
    o_ref[...] = agg + _bn_sc(sc_ref[...], gsc_ref[...], besc_ref[...])


def _knn_finalize(e3r, st3, q3, sc_pre, qsc, ne):
    body = functools.partial(_knn_fin_body, float(ne))
    return _pc(body, jax.ShapeDtypeStruct((N, H), jnp.float32))(
        e3r, st3, q3["g"].reshape(1, H), q3["be"].reshape(1, H),
        sc_pre, qsc["g"].reshape(1, H), qsc["be"].reshape(1, H))


_RB_TK = 400
_CT_TK = 512
_NP_TK = 10240
_BIGI = 1 << 30


def _topk_body(x_ref, d2c_ref, d2r_ref, br_blk_ref, bc_ref, idx_ref):
    i = pl.program_id(0)
    R = _RB_TK
    CT = _CT_TK
    xr = x_ref[pl.ds(i * R, R), :]
    d2r_blk = d2c_ref[pl.ds(i * R, R), :]
    br = br_blk_ref[...]
    b_lo = br[0, 0]
    b_hi = br[R - 1, 0]
    iota_n = lax.broadcasted_iota(jnp.int32, (1, _NP_TK), 1)
    bc_all = bc_ref[...]
    cmin = jnp.min(jnp.where(bc_all[0:1, :] == b_lo, iota_n, _BIGI))
    cmax = jnp.max(jnp.where(bc_all[0:1, :] == b_hi, iota_n, -1)) + 1
    t0 = cmin // CT
    t1 = (cmax + CT - 1) // CT
    row_ids = i * R + lax.broadcasted_iota(jnp.int32, (R, 1), 0)
    inf = jnp.float32(jnp.inf)

    def _sel4(nv, ni):
        bvs = []
        bis = []
        for _ in range(K):
            m = jnp.min(nv, axis=1, keepdims=True)
            mi = jnp.min(jnp.where(nv == m, ni, _BIGI), axis=1, keepdims=True)
            nv = jnp.where(ni == mi, inf, nv)
            bvs.append(m)
            bis.append(mi)
        return jnp.concatenate(bvs, axis=1), jnp.concatenate(bis, axis=1)

    def tile_step(t, carry):
        bv, bi = carry
        c0 = pl.multiple_of(t * CT, CT)
        xc = x_ref[pl.ds(c0, CT), :]
        qk = lax.dot_general(xr, xc, (((1,), (1,)), ((), ())),
                             preferred_element_type=jnp.float32)
        d2c_row = d2r_ref[:, pl.ds(c0, CT)]
        dist = d2r_blk + d2c_row - 2.0 * qk
        bct = bc_ref[:, pl.ds(c0, CT)]
        col_ids = c0 + lax.broadcasted_iota(jnp.int32, (1, CT), 1)
        bad = (bct != br) | (col_ids == row_ids)
        cand = jnp.where(bad, inf, dist)
        tv = []
        ti = []
        for _ in range(K):
            tm = jnp.min(cand, axis=1, keepdims=True)
            tix = jnp.min(jnp.where(cand == tm, col_ids, _BIGI),
                          axis=1, keepdims=True)
            cand = jnp.where(col_ids == tix, inf, cand)
            tv.append(tm)
            ti.append(tix)
        nv = jnp.concatenate([bv] + tv, axis=1)
        ni = jnp.concatenate([bi] + ti, axis=1)
        return _sel4(nv, ni)

    bv0 = jnp.full((R, K), inf, jnp.float32)
    bi0 = jnp.full((R, K), _BIGI, jnp.int32)
    bv, bi = lax.fori_loop(t0, t1, tile_step, (bv0, bi0))

    s0 = t0 * CT
    s1 = jnp.minimum(t1 * CT, N)
    kk = lax.broadcasted_iota(jnp.int32, (1, K), 1)
    ids_out = jnp.where(kk < s0, kk, s1 + kk - s0)
    nv = jnp.concatenate([bv, jnp.full((R, K), inf, jnp.float32)], axis=1)
    ni = jnp.concatenate([bi, jnp.broadcast_to(ids_out, (R, K))], axis=1)
    bv, bi = _sel4(nv, ni)
    idx_ref[...] = bi


def _knn_topk(x_pad, d2c, d2r_pad, br, bc_pad):
    return _pc(
        _topk_body,
        jax.ShapeDtypeStruct((N, K), jnp.int32),
        grid=(N // _RB_TK,),
        in_specs=[
            pl.BlockSpec((_NP_TK, D), lambda i: (0, 0)),
            pl.BlockSpec((N, 1), lambda i: (0, 0)),
            pl.BlockSpec((1, _NP_TK), lambda i: (0, 0)),
            pl.BlockSpec((_RB_TK, 1), lambda i: (i, 0)),
            pl.BlockSpec((1, _NP_TK), lambda i: (0, 0)),
        ],
        out_specs=pl.BlockSpec((_RB_TK, K), lambda i: (i, 0)),
    )(x_pad, d2c, d2r_pad, br, bc_pad)


def _bn_rows(x, g, be):
    m = jnp.mean(x, axis=0, keepdims=True)
    d = x - m
    v = jnp.mean(d * d, axis=0, keepdims=True)
    return d / jnp.sqrt(v + EPS) * g + be


def _head_body(p1_ref, p2_ref, p3_ref, gi_ref, cnt_ref,
               g0a_ref, b0a_ref, g0b_ref, b0b_ref, g0c_ref, b0c_ref,
               g0g_ref, b0g_ref,
               w1a_ref, w1b_ref, w1c_ref, w1g_ref, b1_ref, g1_ref, be1_ref,
               w2_ref, b2_ref, g2_ref, be2_ref, wo_ref, bo_ref, o_ref):
    cnt = jnp.maximum(cnt_ref[...], 1.0)
    p1 = p1_ref[...] / cnt
    p2 = p2_ref[...] / cnt
    p3 = p3_ref[...] / cnt
    gi = gi_ref[...]
    p1 = _bn_rows(p1, g0a_ref[...], b0a_ref[...])
    p2 = _bn_rows(p2, g0b_ref[...], b0b_ref[...])
    p3 = _bn_rows(p3, g0c_ref[...], b0c_ref[...])
    gi = _bn_rows(gi, g0g_ref[...], b0g_ref[...])
    h = (jnp.dot(p1, w1a_ref[...], preferred_element_type=jnp.float32)
         + jnp.dot(p2, w1b_ref[...], preferred_element_type=jnp.float32)
         + jnp.dot(p3, w1c_ref[...], preferred_element_type=jnp.float32)
         + jnp.dot(gi, w1g_ref[...], preferred_element_type=jnp.float32)
         + b1_ref[...])
    h = _bn_rows(_lrelu(h), g1_ref[...], be1_ref[...])
    h = _lrelu(jnp.dot(h, w2_ref[...], preferred_element_type=jnp.float32)
               + b2_ref[...])
    h = _bn_rows(h, g2_ref[...], be2_ref[...])
    lo = jnp.dot(h, wo_ref[...], preferred_element_type=jnp.float32) + bo_ref[...]
    m = jnp.max(lo, axis=1, keepdims=True)
    ex = jnp.exp(lo - m)
    o_ref[...] = ex / jnp.sum(ex, axis=1, keepdims=True)


def _head(c1, c2, c3, gi, bc3, hp):
    p1s, cnt = _segsum(c1, bc3)
    p2s, _ = _segsum(c2, bc3)
    p3s, _ = _segsum(c3, bc3)
    w1t = hp["d1_W"].T
    args = [
        p1s, p2s, p3s, gi, cnt,
        hp["bn0_g"][0:H].reshape(1, H), hp["bn0_b"][0:H].reshape(1, H),
        hp["bn0_g"][H:2 * H].reshape(1, H), hp["bn0_b"][H:2 * H].reshape(1, H),
        hp["bn0_g"][2 * H:3 * H].reshape(1, H), hp["bn0_b"][2 * H:3 * H].reshape(1, H),
        hp["bn0_g"][3 * H:].reshape(1, GF), hp["bn0_b"][3 * H:].reshape(1, GF),
        w1t[0:H], w1t[H:2 * H], w1t[2 * H:3 * H], w1t[3 * H:],
        hp["d1_b"].reshape(1, H), hp["bn1_g"].reshape(1, H), hp["bn1_b"].reshape(1, H),
        hp["d2_W"].T, hp["d2_b"].reshape(1, H),
        hp["bn2_g"].reshape(1, H), hp["bn2_b"].reshape(1, H),
        hp["out_W"].T, hp["out_b"].reshape(1, NC),
    ]
    return _pc(_head_body, jax.ShapeDtypeStruct((G, NC), jnp.float32))(*args)


_CE = 128
_NW = 32
_HALF = 5120
_ACC_R = 6144
_LDUMP = 6136
_DUMP = 10232


def _epad(ne):
    return ((ne + _NW * _CE - 1) // (_NW * _CE)) * (_NW * _CE)


def _edge_gather(x, sd2, ep):
    from jax.experimental.pallas import tpu_sc as plsc
    nch = ep // (_NW * _CE)
    mesh = plsc.VectorSubcoreMesh(core_axis_name="c", subcore_axis_name="s")

    @functools.partial(
        pl.kernel, mesh=mesh,
        out_type=[jax.ShapeDtypeStruct((ep, H), jnp.float32),
                  jax.ShapeDtypeStruct((ep, H), jnp.float32)],
        scratch_types=[
            pltpu.VMEM((2, _CE), jnp.int32),
            pltpu.VMEM((2, _CE), jnp.int32),
            pltpu.VMEM((_CE, H), jnp.float32),
            pltpu.VMEM((_CE, H), jnp.float32),
            pltpu.VMEM((_CE, H), jnp.float32),
            pltpu.VMEM((_CE, H), jnp.float32),
            pltpu.SemaphoreType.DMA,
            pltpu.SemaphoreType.DMA,
            pltpu.SemaphoreType.DMA,
            pltpu.SemaphoreType.DMA,
        ])
    def k(x_hbm, sd_hbm, oi_hbm, oj_hbm,
          ix0, ix1, bi0, bj0, bi1, bj1, sg0, sg1, so0, so1):
        wid = lax.axis_index("s") * 2 + lax.axis_index("c")
        base0 = wid * (ep // _NW)
        ix = (ix0, ix1)
        bi = (bi0, bi1)
        bj = (bj0, bj1)
        sg = (sg0, sg1)
        so = (so0, so1)

        def load_fire(j, s):
            base = base0 + j * _CE
            pltpu.sync_copy(sd_hbm.at[:, pl.ds(base, _CE)], ix[s])
            pltpu.async_copy(x_hbm.at[ix[s].at[0]], bi[s], sg[s])
            pltpu.async_copy(x_hbm.at[ix[s].at[1]], bj[s], sg[s])

        def drain_g(s):
            pltpu.make_async_copy(x_hbm.at[ix[s].at[0]], bi[s], sg[s]).wait()
            pltpu.make_async_copy(x_hbm.at[ix[s].at[1]], bj[s], sg[s]).wait()

        def fire_out(j, s):
            base = base0 + j * _CE
            pltpu.async_copy(bi[s], oi_hbm.at[pl.ds(base, _CE)], so[s])
            pltpu.async_copy(bj[s], oj_hbm.at[pl.ds(base, _CE)], so[s])

        def drain_out(s):
            pltpu.make_async_copy(bi[s], oi_hbm.at[pl.ds(0, _CE)], so[s]).wait()
            pltpu.make_async_copy(bj[s], oj_hbm.at[pl.ds(0, _CE)], so[s]).wait()

        load_fire(0, 0)

        def pair(t2, carry):
            for s in (0, 1):
                j = 2 * t2 + s

                @pl.when(j < nch)
                def _():
                    drain_g(s)

                    @pl.when(j + 1 < nch)
                    def _():
                        @pl.when(j >= 1)
                        def _():
                            drain_out(1 - s)

                        load_fire(j + 1, 1 - s)

                    fire_out(j, s)
            return carry

        lax.fori_loop(0, (nch + 1) // 2, pair, 0)
        drain_out((nch - 1) % 2)
        drain_out(nch % 2)

    return k(x, sd2)


def _edge_scatter(e3, dstp, ne):
    from jax.experimental.pallas import tpu_sc as plsc
    ep = e3.shape[0]
    nch = ep // (16 * _CE)
    mesh = plsc.VectorSubcoreMesh(core_axis_name="c", subcore_axis_name="s")
    rpt = _ACC_R // 16

    @functools.partial(
        pl.kernel, mesh=mesh,
        out_type=[jax.ShapeDtypeStruct((2, _ACC_R, H), jnp.float32),
                  jax.ShapeDtypeStruct((2, _ACC_R, 16), jnp.float32)],
        scratch_types=[
            pltpu.VMEM((_CE,), jnp.int32),
            pltpu.VMEM((1, _CE), jnp.int32),
            pltpu.VMEM((_CE, H), jnp.float32),
            pltpu.VMEM((_CE, 16), jnp.float32),
            pltpu.VMEM((_CE, 16), jnp.float32),
            pltpu.VMEM_SHARED((_ACC_R, H), jnp.float32),
            pltpu.VMEM_SHARED((_ACC_R, 16), jnp.float32),
            pltpu.SemaphoreType.DMA,
        ])
    def k(e_hbm, dst_hbm, acc_hbm, cnt_hbm, di, dl2, rows, ones_v, z16, acc_s,
          cnt_s, sem):
        cid = lax.axis_index("c")
        sid = lax.axis_index("s")
        lo = cid * _HALF

        def zrow(r, c2):
            for g in range(H // 16):
                rows[r, pl.ds(g * 16, 16)] = jnp.zeros((16,), jnp.float32)
            ones_v[r, pl.ds(0, 16)] = jnp.full((16,), 1.0, jnp.float32)
            z16[r, pl.ds(0, 16)] = jnp.zeros((16,), jnp.float32)
            return c2

        lax.fori_loop(0, _CE, zrow, 0)
        for z in range(rpt // _CE):
            pltpu.sync_copy(rows, acc_s.at[pl.ds(sid * rpt + z * _CE, _CE)])
            pltpu.sync_copy(z16, cnt_s.at[pl.ds(sid * rpt + z * _CE, _CE)])
        plsc.subcore_barrier()

        base0 = sid * (ep // 16)

        def chunk(j, carry):
            base = base0 + j * _CE
            pltpu.sync_copy(dst_hbm.at[pl.ds(base, _CE)], di)
            pltpu.async_copy(e_hbm.at[pl.ds(base, _CE)], rows, sem).wait()
            for g in range(_CE // 16):
                dv = di[pl.ds(g * 16, 16)]
                dl = dv - lo
                ok = (dl >= 0) & (dl < _HALF)
                dl2[0, pl.ds(g * 16, 16)] = jnp.where(ok, dl, _LDUMP)
            idxrow = dl2.at[0]
            pltpu.sync_copy(rows, acc_s.at[idxrow], add=True)
            pltpu.sync_copy(ones_v, cnt_s.at[idxrow], add=True)
            return carry

        lax.fori_loop(0, nch, chunk, 0)
        plsc.subcore_barrier()
        for z in range(rpt // _CE):
            r0 = sid * rpt + z * _CE
            pltpu.sync_copy(acc_s.at[pl.ds(r0, _CE)], rows)
            pltpu.sync_copy(rows, acc_hbm.at[cid, pl.ds(r0, _CE)])
            pltpu.sync_copy(cnt_s.at[pl.ds(r0, _CE)], ones_v)
            pltpu.sync_copy(ones_v, cnt_hbm.at[cid, pl.ds(r0, _CE)])

    acc, cnt = k(e3, dstp)
    accf = jnp.concatenate([acc[0, :_HALF], acc[1, :_HALF]], axis=0)[:N]
    cntf = jnp.concatenate([cnt[0, :_HALF], cnt[1, :_HALF]], axis=0)[:N]
    return accf.reshape(1, N, H), cntf[:, 0:1].reshape(1, N, 1)


def _edge_scatter_jnp(e3, dstp, ne):
    acc = jax.ops.segment_sum(e3[:ne], dstp[:ne], num_segments=N)
    cnt = jax.ops.segment_sum(jnp.ones((ne,), jnp.float32), dstp[:ne],
                              num_segments=N)
    return acc.reshape(1, N, H), cnt.reshape(1, N, 1)


def _edge_mlp(xi, xj, q, ne):
    e1, st1 = _mlp_pass1(xi, xj, q["m1"]["W"].T, q["m1"]["b"], ne)
    e2, st2 = _mlp_pass(e1, st1, q["m2"]["W"].T, q["m2"]["b"],
                        q["m1"]["g"], q["m1"]["be"], ne)
    e3, st3 = _mlp_pass(e2, st2, q["m3"]["W"].T, q["m3"]["b"],
                        q["m2"]["g"], q["m2"]["be"], ne)
    return e3, st3


def _edge_conv(xin, src, dst, q):
    sc_pre = _node_mm(xin, q["sc"]["W"].T, q["sc"]["b"])
    ep = _epad(E)
    pad = ep - E
    zpad = jnp.zeros((pad,), jnp.int32)
    srcp = jnp.concatenate([src, zpad])
    dstg = jnp.concatenate([dst, zpad])
    dsts = jnp.concatenate([dst, jnp.full((pad,), _DUMP, jnp.int32)])
    xi, xj = _edge_gather(xin, jnp.stack([dstg, srcp]), ep)
    e3, st3 = _edge_mlp(xi, xj, q, E)
    acc, cnt = _edge_scatter_jnp(e3, dsts, E)
    return _econv_finalize(acc, cnt, st3, q["m3"], sc_pre, q["sc"], E)


def _knn_conv(xin, br, bc, dstk, q):
    sc_pre = _node_mm(xin, q["sc"]["W"].T, q["sc"]["b"])
    d2 = jnp.sum(xin * xin, axis=1)
    x_pad = jnp.concatenate(
        [xin, jnp.zeros((_NP_TK - N, D), jnp.float32)], axis=0)
    d2r_pad = jnp.concatenate(
        [d2, jnp.zeros((_NP_TK - N,), jnp.float32)]).reshape(1, _NP_TK)
    bc_pad = jnp.concatenate(
        [bc[0], jnp.full((_NP_TK - N,), -1, jnp.int32)]).reshape(1, _NP_TK)
    idx = _knn_topk(x_pad, d2.reshape(N, 1), d2r_pad, br, bc_pad)
    nk = N * K
    ep = _epad(nk)
    zpad = jnp.zeros((ep - nk,), jnp.int32)
    srck = jnp.concatenate([idx.reshape(-1), zpad])
    dstkp = jnp.concatenate([dstk, zpad])
    xi, xj = _edge_gather(xin, jnp.stack([dstkp, srck]), ep)
    e3, st3 = _edge_mlp(xi, xj, q, nk)
    return _knn_finalize(e3[:nk].reshape(N, K * H), st3, q["m3"],
                         sc_pre, q["sc"], nk)


def kernel(x, edge_index, graph_input, batch, params):
    br = batch.reshape(N, 1)
    bc = batch.reshape(1, N)
    src = edge_index[0]
    dst = edge_index[1]
    dstk = jnp.repeat(jnp.arange(N, dtype=jnp.int32), K)

    bc3 = batch.reshape(N // _RB_N, 1, _RB_N)
    xg = _graph_norm(x, br, bc3, params["gn"])
    c1 = _edge_conv(xg, src, dst, params["conv1"])
    c2 = _knn_conv(c1, br, bc, dstk, params["conv2"])
    c3 = _knn_conv(c2, br, bc, dstk, params["conv3"])
    return _head(c1, c2, c3, graph_input, bc3, params["head"])


# speedup vs baseline: 1.3007x; 1.0703x over previous
import functools

import jax
import jax.numpy as jnp
from jax import lax
from jax.experimental import pallas as pl
from jax.experimental.pallas import tpu as pltpu

N = 10000
E = 320000
D = 128
H = 128
G = 256
GF = 4
NC = 2
K = 4
EPS = 1e-5

_HI = lax.Precision.HIGHEST
_INTERPRET = False


def _lrelu(x):
    return jnp.where(x >= 0, x, 0.01 * x)


def _pc(body, out_shape, grid=None, in_specs=None, out_specs=None, scratch_shapes=None):
    kw = {}
    if grid is not None:
        kw["grid"] = grid
    if in_specs is not None:
        kw["in_specs"] = in_specs
    if out_specs is not None:
        kw["out_specs"] = out_specs
    if scratch_shapes is not None:
        kw["scratch_shapes"] = scratch_shapes
    return pl.pallas_call(body, out_shape=out_shape, interpret=_INTERPRET, **kw)


_RB_N = 2000


def _segsum_body(y_ref, bc_ref, s_ref, c_ref, accs, accc):
    i = pl.program_id(0)
    iota_col = lax.broadcasted_iota(jnp.int32, (G, 1), 0)
    MT = (iota_col == bc_ref[0]).astype(jnp.float32)
    ps = jnp.dot(MT, y_ref[...], precision=_HI, preferred_element_type=jnp.float32)
    pc_ = jnp.sum(MT, axis=1, keepdims=True)

    @pl.when(i == 0)
    def _():
        accs[...] = jnp.zeros_like(accs)
        accc[...] = jnp.zeros_like(accc)

    accs[...] += ps
    accc[...] += pc_

    @pl.when(i == pl.num_programs(0) - 1)
    def _():
        s_ref[...] = accs[...]
        c_ref[...] = accc[...]


def _segsum(y, bc3):
    c = y.shape[1]
    return _pc(
        _segsum_body,
        (jax.ShapeDtypeStruct((G, c), jnp.float32),
         jax.ShapeDtypeStruct((G, 1), jnp.float32)),
        grid=(N // _RB_N,),
        in_specs=[pl.BlockSpec((_RB_N, c), lambda i: (i, 0)),
                  pl.BlockSpec((1, 1, _RB_N), lambda i: (i, 0, 0))],
        out_specs=(pl.BlockSpec((G, c), lambda i: (0, 0)),
                   pl.BlockSpec((G, 1), lambda i: (0, 0))),
        scratch_shapes=[pltpu.VMEM((G, c), jnp.float32),
                        pltpu.VMEM((G, 1), jnp.float32)],
    )(y, bc3)


def _gn_mid_body(x_ref, br_ref, bc_ref, s_ref, c_ref, ms_ref,
                 o_ref, v_ref, accv):
    i = pl.program_id(0)
    cnt = jnp.maximum(c_ref[...], 1.0)
    mean = s_ref[...] / cnt
    iota_row = lax.broadcasted_iota(jnp.int32, (1, G), 1)
    M = (br_ref[...] == iota_row).astype(jnp.float32)
    meanb = jnp.dot(M, mean, precision=_HI, preferred_element_type=jnp.float32)
    out0 = x_ref[...] - meanb * ms_ref[...]
    o_ref[...] = out0
    iota_col = lax.broadcasted_iota(jnp.int32, (G, 1), 0)
    MT = (iota_col == bc_ref[0]).astype(jnp.float32)
    pv = jnp.dot(MT, out0 * out0, precision=_HI,
                 preferred_element_type=jnp.float32)

    @pl.when(i == 0)
    def _():
        accv[...] = jnp.zeros_like(accv)

    accv[...] += pv

    @pl.when(i == pl.num_programs(0) - 1)
    def _():
        v_ref[...] = accv[...]


def _gn_fin_body(o0_ref, br_ref, v_ref, c_ref, w_ref, b_ref, o_ref):
    cnt = jnp.maximum(c_ref[...], 1.0)
    var = v_ref[...] / cnt
    iota_row = lax.broadcasted_iota(jnp.int32, (1, G), 1)
    M = (br_ref[...] == iota_row).astype(jnp.float32)
    varb = jnp.dot(M, var, precision=_HI, preferred_element_type=jnp.float32)
    o_ref[...] = w_ref[...] * o0_ref[...] / jnp.sqrt(varb + EPS) + b_ref[...]


def _graph_norm(x, br, bc3, p):
    sums, cnt = _segsum(x, bc3)
    out0, vsums = _pc(
        _gn_mid_body,
        (jax.ShapeDtypeStruct((N, D), jnp.float32),
         jax.ShapeDtypeStruct((G, D), jnp.float32)),
        grid=(N // _RB_N,),
        in_specs=[pl.BlockSpec((_RB_N, D), lambda i: (i, 0)),
                  pl.BlockSpec((_RB_N, 1), lambda i: (i, 0)),
                  pl.BlockSpec((1, 1, _RB_N), lambda i: (i, 0, 0)),
                  pl.BlockSpec((G, D), lambda i: (0, 0)),
                  pl.BlockSpec((G, 1), lambda i: (0, 0)),
                  pl.BlockSpec((1, D), lambda i: (0, 0))],
        out_specs=(pl.BlockSpec((_RB_N, D), lambda i: (i, 0)),
                   pl.BlockSpec((G, D), lambda i: (0, 0))),
        scratch_shapes=[pltpu.VMEM((G, D), jnp.float32)],
    )(x, br, bc3, sums, cnt, p["ms"].reshape(1, D))
    return _pc(
        _gn_fin_body,
        jax.ShapeDtypeStruct((N, D), jnp.float32),
        grid=(N // _RB_N,),
        in_specs=[pl.BlockSpec((_RB_N, D), lambda i: (i, 0)),
                  pl.BlockSpec((_RB_N, 1), lambda i: (i, 0)),
                  pl.BlockSpec((G, D), lambda i: (0, 0)),
                  pl.BlockSpec((G, 1), lambda i: (0, 0)),
                  pl.BlockSpec((1, D), lambda i: (0, 0)),
                  pl.BlockSpec((1, D), lambda i: (0, 0))],
        out_specs=pl.BlockSpec((_RB_N, D), lambda i: (i, 0)),
    )(out0, br, vsums, cnt, p["w"].reshape(1, D), p["b"].reshape(1, D))


def _mm_body(x_ref, w_ref, b_ref, o_ref):
    o_ref[...] = jnp.dot(x_ref[...], w_ref[...],
                         preferred_element_type=jnp.float32) + b_ref[...]


def _node_mm(x, wt, b):
    n, _ = x.shape
    co = wt.shape[1]
    return _pc(_mm_body, jax.ShapeDtypeStruct((n, co), jnp.float32))(
        x, wt, b.reshape(1, co))


_RB_E = 4096


def _mlp1_body(ne, xi_ref, xj_ref, w_ref, b_ref, o_ref, so_ref, acc):
    i = pl.program_id(0)
    xi = xi_ref[...]
    cat = jnp.concatenate([xi, xj_ref[...] - xi], axis=1)
    out = _lrelu(jnp.dot(cat, w_ref[...],
                         preferred_element_type=jnp.float32) + b_ref[...])
    o_ref[...] = out

    @pl.when(i == 0)
    def _():
        acc[...] = jnp.zeros_like(acc)

    rb = out.shape[0]
    gid = i * rb + lax.broadcasted_iota(jnp.int32, (rb, 1), 0)
    outm = jnp.where(gid < ne, out, 0.0)
    acc[0:1, :] += jnp.sum(outm, axis=0, keepdims=True)
    acc[1:2, :] += jnp.sum(outm * outm, axis=0, keepdims=True)

    @pl.when(i == pl.num_programs(0) - 1)
    def _():
        so_ref[...] = acc[...]


def _mlp_pass1(xi, xj, wt, b, ne):
    ep = xi.shape[0]
    nblk = ep // _RB_E
    body = functools.partial(_mlp1_body, ne)
    return _pc(
        body,
        (jax.ShapeDtypeStruct((ep, H), jnp.float32),
         jax.ShapeDtypeStruct((2, H), jnp.float32)),
        grid=(nblk,),
        in_specs=[
            pl.BlockSpec((_RB_E, H), lambda i: (i, 0)),
            pl.BlockSpec((_RB_E, H), lambda i: (i, 0)),
            pl.BlockSpec((2 * H, H), lambda i: (0, 0)),
            pl.BlockSpec((1, H), lambda i: (0, 0)),
        ],
        out_specs=(pl.BlockSpec((_RB_E, H), lambda i: (i, 0)),
                   pl.BlockSpec((2, H), lambda i: (0, 0))),
        scratch_shapes=[pltpu.VMEM((2, H), jnp.float32)],
    )(xi, xj, wt, b.reshape(1, H))


def _mlp1k_body(ne, xn_ref, xj_ref, w_ref, b_ref, o_ref, so_ref, acc):
    i = pl.program_id(0)
    xn = xn_ref[...]
    rbn = xn.shape[0]
    xi = jnp.reshape(jnp.broadcast_to(xn[:, None, :], (rbn, K, H)),
                     (rbn * K, H))
    cat = jnp.concatenate([xi, xj_ref[...] - xi], axis=1)
    out = _lrelu(jnp.dot(cat, w_ref[...],
                         preferred_element_type=jnp.float32) + b_ref[...])
    o_ref[...] = out

    @pl.when(i == 0)
    def _():
        acc[...] = jnp.zeros_like(acc)

    rb = out.shape[0]
    gid = i * rb + lax.broadcasted_iota(jnp.int32, (rb, 1), 0)
    outm = jnp.where(gid < ne, out, 0.0)
    acc[0:1, :] += jnp.sum(outm, axis=0, keepdims=True)
    acc[1:2, :] += jnp.sum(outm * outm, axis=0, keepdims=True)

    @pl.when(i == pl.num_programs(0) - 1)
    def _():
        so_ref[...] = acc[...]


def _mlp_pass1k(x_pad, xj, wt, b, ne):
    ep = xj.shape[0]
    nblk = ep // _RB_E
    body = functools.partial(_mlp1k_body, ne)
    return _pc(
        body,
        (jax.ShapeDtypeStruct((ep, H), jnp.float32),
         jax.ShapeDtypeStruct((2, H), jnp.float32)),
        grid=(nblk,),
        in_specs=[
            pl.BlockSpec((_RB_E // K, H), lambda i: (i, 0)),
            pl.BlockSpec((_RB_E, H), lambda i: (i, 0)),
            pl.BlockSpec((2 * H, H), lambda i: (0, 0)),
            pl.BlockSpec((1, H), lambda i: (0, 0)),
        ],
        out_specs=(pl.BlockSpec((_RB_E, H), lambda i: (i, 0)),
                   pl.BlockSpec((2, H), lambda i: (0, 0))),
        scratch_shapes=[pltpu.VMEM((2, H), jnp.float32)],
    )(x_pad, xj, wt, b.reshape(1, H))


def _mlp_pass_body(ne, e_ref, st_ref, w_ref, bcur_ref, gprev_ref, beprev_ref,
                   o_ref, so_ref, acc):
    i = pl.program_id(0)
    st = st_ref[...]
    mean = st[0:1, :] / ne
    var = st[1:2, :] / ne - mean * mean
    en = (e_ref[...] - mean) / jnp.sqrt(var + EPS) * gprev_ref[...] + beprev_ref[...]
    out = _lrelu(jnp.dot(en, w_ref[...], preferred_element_type=jnp.float32)
                 + bcur_ref[...])
    o_ref[...] = out

    @pl.when(i == 0)
    def _():
        acc[...] = jnp.zeros_like(acc)

    rb = out.shape[0]
    gid = i * rb + lax.broadcasted_iota(jnp.int32, (rb, 1), 0)
    outm = jnp.where(gid < ne, out, 0.0)
    acc[0:1, :] += jnp.sum(outm, axis=0, keepdims=True)
    acc[1:2, :] += jnp.sum(outm * outm, axis=0, keepdims=True)

    @pl.when(i == pl.num_programs(0) - 1)
    def _():
        so_ref[...] = acc[...]


def _mlp_pass(e, stats, wt, b_cur, g_prev, be_prev, ne):
    ep = e.shape[0]
    nblk = ep // _RB_E
    body = functools.partial(_mlp_pass_body, float(ne))
    return _pc(
        body,
        (jax.ShapeDtypeStruct((ep, H), jnp.float32),
         jax.ShapeDtypeStruct((2, H), jnp.float32)),
        grid=(nblk,),
        in_specs=[
            pl.BlockSpec((_RB_E, H), lambda i: (i, 0)),
            pl.BlockSpec((2, H), lambda i: (0, 0)),
            pl.BlockSpec((H, H), lambda i: (0, 0)),
            pl.BlockSpec((1, H), lambda i: (0, 0)),
            pl.BlockSpec((1, H), lambda i: (0, 0)),
            pl.BlockSpec((1, H), lambda i: (0, 0)),
        ],
        out_specs=(pl.BlockSpec((_RB_E, H), lambda i: (i, 0)),
                   pl.BlockSpec((2, H), lambda i: (0, 0))),
        scratch_shapes=[pltpu.VMEM((2, H), jnp.float32)],
    )(e, stats, wt, b_cur.reshape(1, H), g_prev.reshape(1, H), be_prev.reshape(1, H))


def _bn_sc(sc, g, be):
    m = jnp.mean(sc, axis=0, keepdims=True)
    d = sc - m
    v = jnp.mean(d * d, axis=0, keepdims=True)
    return d / jnp.sqrt(v + EPS) * g + be


def _econv_fin_body(ne, acc_ref, cnt_ref, st3_ref, g3_ref, be3_ref,
                    sc_ref, gsc_ref, besc_ref, o_ref):
    st = st3_ref[...]
    mean3 = st[0:1, :] / ne
    var3 = st[1:2, :] / ne - mean3 * mean3
    s3 = g3_ref[...] / jnp.sqrt(var3 + EPS)
    c3 = be3_ref[...] - mean3 * s3
    accs = jnp.sum(acc_ref[...], axis=0)
    cnt = jnp.sum(cnt_ref[...], axis=0)
    agg = (accs * s3 + c3 * cnt) / jnp.maximum(cnt, 1.0)
    o_ref[...] = agg + _bn_sc(sc_ref[...], gsc_ref[...], besc_ref[...])


def _econv_finalize(acc, cnt, st3, q3, sc_pre, qsc, ne):
    body = functools.partial(_econv_fin_body, float(ne))
    return _pc(body, jax.ShapeDtypeStruct((N, H), jnp.float32))(
        acc, cnt, st3, q3["g"].reshape(1, H), q3["be"].reshape(1, H),
        sc_pre, qsc["g"].reshape(1, H), qsc["be"].reshape(1, H))


def _knn_fin_body(ne, e_ref, st3_ref, g3_ref, be3_ref, sc_ref, gsc_ref,
                  besc_ref, o_ref):
    st = st3_ref[...]
    mean3 = st[0:1, :] / ne
    var3 = st[1:2, :] / ne - mean3 * mean3
    s3 = g3_ref[...] / jnp.sqrt(var3 + EPS)
    c3 = be3_ref[...] - mean3 * s3
    e = e_ref[...]
    mn = (e[:, 0:H] + e[:, H:2 * H] + e[:, 2 * H:3 * H] + e[:, 3 * H:4 * H]) * 0.25
    agg = mn * s3 + c3
    o_ref[...] = agg + _bn_sc(sc_ref[...], gsc_ref[...], besc_ref[...])


def _knn_finalize(e3r, st3, q3, sc_pre, qsc, ne):
    body = functools.partial(_knn_fin_body, float(ne))
    return _pc(body, jax.ShapeDtypeStruct((N, H), jnp.float32))(
        e3r, st3, q3["g"].reshape(1, H), q3["be"].reshape(1, H),
        sc_pre, qsc["g"].reshape(1, H), qsc["be"].reshape(1, H))


_RB_TK = 400
_CT_TK = 512
_NP_TK = 10240
_BIGI = 1 << 30


def _topk_body(x_ref, d2c_ref, d2r_ref, br_blk_ref, bc_ref, idx_ref):
    i = pl.program_id(0)
    R = _RB_TK
    CT = _CT_TK
    xr = x_ref[pl.ds(i * R, R), :]
    d2r_blk = d2c_ref[pl.ds(i * R, R), :]
    br = br_blk_ref[...]
    b_lo = br[0, 0]
    b_hi = br[R - 1, 0]
    iota_n = lax.broadcasted_iota(jnp.int32, (1, _NP_TK), 1)
    bc_all = bc_ref[...]
    cmin = jnp.min(jnp.where(bc_all[0:1, :] == b_lo, iota_n, _BIGI))
    cmax = jnp.max(jnp.where(bc_all[0:1, :] == b_hi, iota_n, -1)) + 1
    t0 = cmin // CT
    t1 = (cmax + CT - 1) // CT
    row_ids = i * R + lax.broadcasted_iota(jnp.int32, (R, 1), 0)
    inf = jnp.float32(jnp.inf)

    def _sel4(nv, ni):
        bvs = []
        bis = []
        for _ in range(K):
            m = jnp.min(nv, axis=1, keepdims=True)
            mi = jnp.min(jnp.where(nv == m, ni, _BIGI), axis=1, keepdims=True)
            nv = jnp.where(ni == mi, inf, nv)
            bvs.append(m)
            bis.append(mi)
        return jnp.concatenate(bvs, axis=1), jnp.concatenate(bis, axis=1)

    def tile_step(t, carry):
        bv, bi = carry
        c0 = pl.multiple_of(t * CT, CT)
        xc = x_ref[pl.ds(c0, CT), :]
        qk = lax.dot_general(xr, xc, (((1,), (1,)), ((), ())),
                             preferred_element_type=jnp.float32)
        d2c_row = d2r_ref[:, pl.ds(c0, CT)]
        dist = d2r_blk + d2c_row - 2.0 * qk
        bct = bc_ref[:, pl.ds(c0, CT)]
        col_ids = c0 + lax.broadcasted_iota(jnp.int32, (1, CT), 1)
        bad = (bct != br) | (col_ids == row_ids)
        cand = jnp.where(bad, inf, dist)
        tv = []
        ti = []
        for _ in range(K):
            tm = jnp.min(cand, axis=1, keepdims=True)
            tix = jnp.min(jnp.where(cand == tm, col_ids, _BIGI),
                          axis=1, keepdims=True)
            cand = jnp.where(col_ids == tix, inf, cand)
            tv.append(tm)
            ti.append(tix)
        nv = jnp.concatenate([bv] + tv, axis=1)
        ni = jnp.concatenate([bi] + ti, axis=1)
        return _sel4(nv, ni)

    bv0 = jnp.full((R, K), inf, jnp.float32)
    bi0 = jnp.full((R, K), _BIGI, jnp.int32)
    bv, bi = lax.fori_loop(t0, t1, tile_step, (bv0, bi0))

    s0 = t0 * CT
    s1 = jnp.minimum(t1 * CT, N)
    kk = lax.broadcasted_iota(jnp.int32, (1, K), 1)
    ids_out = jnp.where(kk < s0, kk, s1 + kk - s0)
    nv = jnp.concatenate([bv, jnp.full((R, K), inf, jnp.float32)], axis=1)
    ni = jnp.concatenate([bi, jnp.broadcast_to(ids_out, (R, K))], axis=1)
    bv, bi = _sel4(nv, ni)
    idx_ref[...] = bi


def _knn_topk(x_pad, d2c, d2r_pad, br, bc_pad):
    return _pc(
        _topk_body,
        jax.ShapeDtypeStruct((N, K), jnp.int32),
        grid=(N // _RB_TK,),
        in_specs=[
            pl.BlockSpec((_NP_TK, D), lambda i: (0, 0)),
            pl.BlockSpec((N, 1), lambda i: (0, 0)),
            pl.BlockSpec((1, _NP_TK), lambda i: (0, 0)),
            pl.BlockSpec((_RB_TK, 1), lambda i: (i, 0)),
            pl.BlockSpec((1, _NP_TK), lambda i: (0, 0)),
        ],
        out_specs=pl.BlockSpec((_RB_TK, K), lambda i: (i, 0)),
    )(x_pad, d2c, d2r_pad, br, bc_pad)


def _bn_rows(x, g, be):
    m = jnp.mean(x, axis=0, keepdims=True)
    d = x - m
    v = jnp.mean(d * d, axis=0, keepdims=True)
    return d / jnp.sqrt(v + EPS) * g + be


def _head_body(p1_ref, p2_ref, p3_ref, gi_ref, cnt_ref,
               g0a_ref, b0a_ref, g0b_ref, b0b_ref, g0c_ref, b0c_ref,
               g0g_ref, b0g_ref,
               w1a_ref, w1b_ref, w1c_ref, w1g_ref, b1_ref, g1_ref, be1_ref,
               w2_ref, b2_ref, g2_ref, be2_ref, wo_ref, bo_ref, o_ref):
    cnt = jnp.maximum(cnt_ref[...], 1.0)
    p1 = p1_ref[...] / cnt
    p2 = p2_ref[...] / cnt
    p3 = p3_ref[...] / cnt
    gi = gi_ref[...]
    p1 = _bn_rows(p1, g0a_ref[...], b0a_ref[...])
    p2 = _bn_rows(p2, g0b_ref[...], b0b_ref[...])
    p3 = _bn_rows(p3, g0c_ref[...], b0c_ref[...])
    gi = _bn_rows(gi, g0g_ref[...], b0g_ref[...])
    h = (jnp.dot(p1, w1a_ref[...], preferred_element_type=jnp.float32)
         + jnp.dot(p2, w1b_ref[...], preferred_element_type=jnp.float32)
         + jnp.dot(p3, w1c_ref[...], preferred_element_type=jnp.float32)
         + jnp.dot(gi, w1g_ref[...], preferred_element_type=jnp.float32)
         + b1_ref[...])
    h = _bn_rows(_lrelu(h), g1_ref[...], be1_ref[...])
    h = _lrelu(jnp.dot(h, w2_ref[...], preferred_element_type=jnp.float32)
               + b2_ref[...])
    h = _bn_rows(h, g2_ref[...], be2_ref[...])
    lo = jnp.dot(h, wo_ref[...], preferred_element_type=jnp.float32) + bo_ref[...]
    m = jnp.max(lo, axis=1, keepdims=True)
    ex = jnp.exp(lo - m)
    o_ref[...] = ex / jnp.sum(ex, axis=1, keepdims=True)


def _head(c1, c2, c3, gi, bc3, hp):
    p1s, cnt = _segsum(c1, bc3)
    p2s, _ = _segsum(c2, bc3)
    p3s, _ = _segsum(c3, bc3)
    w1t = hp["d1_W"].T
    args = [
        p1s, p2s, p3s, gi, cnt,
        hp["bn0_g"][0:H].reshape(1, H), hp["bn0_b"][0:H].reshape(1, H),
        hp["bn0_g"][H:2 * H].reshape(1, H), hp["bn0_b"][H:2 * H].reshape(1, H),
        hp["bn0_g"][2 * H:3 * H].reshape(1, H), hp["bn0_b"][2 * H:3 * H].reshape(1, H),
        hp["bn0_g"][3 * H:].reshape(1, GF), hp["bn0_b"][3 * H:].reshape(1, GF),
        w1t[0:H], w1t[H:2 * H], w1t[2 * H:3 * H], w1t[3 * H:],
        hp["d1_b"].reshape(1, H), hp["bn1_g"].reshape(1, H), hp["bn1_b"].reshape(1, H),
        hp["d2_W"].T, hp["d2_b"].reshape(1, H),
        hp["bn2_g"].reshape(1, H), hp["bn2_b"].reshape(1, H),
        hp["out_W"].T, hp["out_b"].reshape(1, NC),
    ]
    return _pc(_head_body, jax.ShapeDtypeStruct((G, NC), jnp.float32))(*args)


_CE = 128
_NW = 32
_HALF = 5120
_ACC_R = 6144
_LDUMP = 6136
_DUMP = 10232


def _epad(ne):
    return ((ne + _NW * _CE - 1) // (_NW * _CE)) * (_NW * _CE)


def _edge_gather(x, sd2, ep):
    from jax.experimental.pallas import tpu_sc as plsc
    nch = ep // (_NW * _CE)
    mesh = plsc.VectorSubcoreMesh(core_axis_name="c", subcore_axis_name="s")

    @functools.partial(
        pl.kernel, mesh=mesh,
        out_type=[jax.ShapeDtypeStruct((ep, H), jnp.float32),
                  jax.ShapeDtypeStruct((ep, H), jnp.float32)],
        scratch_types=[
            pltpu.VMEM((2, _CE), jnp.int32),
            pltpu.VMEM((2, _CE), jnp.int32),
            pltpu.VMEM((_CE, H), jnp.float32),
            pltpu.VMEM((_CE, H), jnp.float32),
            pltpu.VMEM((_CE, H), jnp.float32),
            pltpu.VMEM((_CE, H), jnp.float32),
            pltpu.SemaphoreType.DMA,
            pltpu.SemaphoreType.DMA,
            pltpu.SemaphoreType.DMA,
            pltpu.SemaphoreType.DMA,
        ])
    def k(x_hbm, sd_hbm, oi_hbm, oj_hbm,
          ix0, ix1, bi0, bj0, bi1, bj1, sg0, sg1, so0, so1):
        wid = lax.axis_index("s") * 2 + lax.axis_index("c")
        base0 = wid * (ep // _NW)
        ix = (ix0, ix1)
        bi = (bi0, bi1)
        bj = (bj0, bj1)
        sg = (sg0, sg1)
        so = (so0, so1)

        def load_fire(j, s):
            base = base0 + j * _CE
            pltpu.sync_copy(sd_hbm.at[:, pl.ds(base, _CE)], ix[s])
            pltpu.async_copy(x_hbm.at[ix[s].at[0]], bi[s], sg[s])
            pltpu.async_copy(x_hbm.at[ix[s].at[1]], bj[s], sg[s])

        def drain_g(s):
            pltpu.make_async_copy(x_hbm.at[ix[s].at[0]], bi[s], sg[s]).wait()
            pltpu.make_async_copy(x_hbm.at[ix[s].at[1]], bj[s], sg[s]).wait()

        def fire_out(j, s):
            base = base0 + j * _CE
            pltpu.async_copy(bi[s], oi_hbm.at[pl.ds(base, _CE)], so[s])
            pltpu.async_copy(bj[s], oj_hbm.at[pl.ds(base, _CE)], so[s])

        def drain_out(s):
            pltpu.make_async_copy(bi[s], oi_hbm.at[pl.ds(0, _CE)], so[s]).wait()
            pltpu.make_async_copy(bj[s], oj_hbm.at[pl.ds(0, _CE)], so[s]).wait()

        load_fire(0, 0)

        def pair(t2, carry):
            for s in (0, 1):
                j = 2 * t2 + s

                @pl.when(j < nch)
                def _():
                    drain_g(s)

                    @pl.when(j + 1 < nch)
                    def _():
                        @pl.when(j >= 1)
                        def _():
                            drain_out(1 - s)

                        load_fire(j + 1, 1 - s)

                    fire_out(j, s)
            return carry

        lax.fori_loop(0, (nch + 1) // 2, pair, 0)
        drain_out((nch - 1) % 2)
        drain_out(nch % 2)

    return k(x, sd2)


def _gather1(x, idxp, ep):
    from jax.experimental.pallas import tpu_sc as plsc
    nch = ep // (_NW * _CE)
    mesh = plsc.VectorSubcoreMesh(core_axis_name="c", subcore_axis_name="s")

    @functools.partial(
        pl.kernel, mesh=mesh,
        out_type=jax.ShapeDtypeStruct((ep, H), jnp.float32),
        scratch_types=[
            pltpu.VMEM((1, _CE), jnp.int32),
            pltpu.VMEM((1, _CE), jnp.int32),
            pltpu.VMEM((_CE, H), jnp.float32),
            pltpu.VMEM((_CE, H), jnp.float32),
            pltpu.SemaphoreType.DMA,
            pltpu.SemaphoreType.DMA,
            pltpu.SemaphoreType.DMA,
            pltpu.SemaphoreType.DMA,
        ])
    def k(x_hbm, idx_hbm, o_hbm, ix0, ix1, b0, b1, sg0, sg1, so0, so1):
        wid = lax.axis_index("s") * 2 + lax.axis_index("c")
        base0 = wid * (ep // _NW)
        ix = (ix0, ix1)
        bb = (b0, b1)
        sg = (sg0, sg1)
        so = (so0, so1)

        def load_fire(j, s):
            base = base0 + j * _CE
            pltpu.sync_copy(idx_hbm.at[:, pl.ds(base, _CE)], ix[s])
            pltpu.async_copy(x_hbm.at[ix[s].at[0]], bb[s], sg[s])

        def drain_g(s):
            pltpu.make_async_copy(x_hbm.at[ix[s].at[0]], bb[s], sg[s]).wait()

        def fire_out(j, s):
            base = base0 + j * _CE
            pltpu.async_copy(bb[s], o_hbm.at[pl.ds(base, _CE)], so[s])

        def drain_out(s):
            pltpu.make_async_copy(bb[s], o_hbm.at[pl.ds(0, _CE)], so[s]).wait()

        load_fire(0, 0)

        def pair(t2, carry):
            for s in (0, 1):
                j = 2 * t2 + s

                @pl.when(j < nch)
                def _():
                    drain_g(s)

                    @pl.when(j + 1 < nch)
                    def _():
                        @pl.when(j >= 1)
                        def _():
                            drain_out(1 - s)

                        load_fire(j + 1, 1 - s)

                    fire_out(j, s)
            return carry

        lax.fori_loop(0, (nch + 1) // 2, pair, 0)
        drain_out((nch - 1) % 2)
        drain_out(nch % 2)

    return k(x, idxp)


def _edge_scatter(e3, dstp, ne):
    from jax.experimental.pallas import tpu_sc as plsc
    ep = e3.shape[0]
    nch = ep // (16 * _CE)
    mesh = plsc.VectorSubcoreMesh(core_axis_name="c", subcore_axis_name="s")
    rpt = _ACC_R // 16

    @functools.partial(
        pl.kernel, mesh=mesh,
        out_type=[jax.ShapeDtypeStruct((2, _ACC_R, H), jnp.float32),
                  jax.ShapeDtypeStruct((2, _ACC_R, 16), jnp.float32)],
        scratch_types=[
            pltpu.VMEM((_CE,), jnp.int32),
            pltpu.VMEM((1, _CE), jnp.int32),
            pltpu.VMEM((_CE, H), jnp.float32),
            pltpu.VMEM((_CE, 16), jnp.float32),
            pltpu.VMEM((_CE, 16), jnp.float32),
            pltpu.VMEM_SHARED((_ACC_R, H), jnp.float32),
            pltpu.VMEM_SHARED((_ACC_R, 16), jnp.float32),
            pltpu.SemaphoreType.DMA,
        ])
    def k(e_hbm, dst_hbm, acc_hbm, cnt_hbm, di, dl2, rows, ones_v, z16, acc_s,
          cnt_s, sem):
        cid = lax.axis_index("c")
        sid = lax.axis_index("s")
        lo = cid * _HALF

        def zrow(r, c2):
            for g in range(H // 16):
                rows[r, pl.ds(g * 16, 16)] = jnp.zeros((16,), jnp.float32)
            ones_v[r, pl.ds(0, 16)] = jnp.full((16,), 1.0, jnp.float32)
            z16[r, pl.ds(0, 16)] = jnp.zeros((16,), jnp.float32)
            return c2

        lax.fori_loop(0, _CE, zrow, 0)
        for z in range(rpt // _CE):
            pltpu.sync_copy(rows, acc_s.at[pl.ds(sid * rpt + z * _CE, _CE)])
            pltpu.sync_copy(z16, cnt_s.at[pl.ds(sid * rpt + z * _CE, _CE)])
        plsc.subcore_barrier()

        base0 = sid * (ep // 16)

        def chunk(j, carry):
            base = base0 + j * _CE
            pltpu.sync_copy(dst_hbm.at[pl.ds(base, _CE)], di)
            pltpu.async_copy(e_hbm.at[pl.ds(base, _CE)], rows, sem).wait()
            for g in range(_CE // 16):
                dv = di[pl.ds(g * 16, 16)]
                dl = dv - lo
                ok = (dl >= 0) & (dl < _HALF)
                dl2[0, pl.ds(g * 16, 16)] = jnp.where(ok, dl, _LDUMP)
            idxrow = dl2.at[0]
            pltpu.sync_copy(rows, acc_s.at[idxrow], add=True)
            pltpu.sync_copy(ones_v, cnt_s.at[idxrow], add=True)
            return carry

        lax.fori_loop(0, nch, chunk, 0)
        plsc.subcore_barrier()
        for z in range(rpt // _CE):
            r0 = sid * rpt + z * _CE
            pltpu.sync_copy(acc_s.at[pl.ds(r0, _CE)], rows)
            pltpu.sync_copy(rows, acc_hbm.at[cid, pl.ds(r0, _CE)])
            pltpu.sync_copy(cnt_s.at[pl.ds(r0, _CE)], ones_v)
            pltpu.sync_copy(ones_v, cnt_hbm.at[cid, pl.ds(r0, _CE)])

    acc, cnt = k(e3, dstp)
    accf = jnp.concatenate([acc[0, :_HALF], acc[1, :_HALF]], axis=0)[:N]
    cntf = jnp.concatenate([cnt[0, :_HALF], cnt[1, :_HALF]], axis=0)[:N]
    return accf.reshape(1, N, H), cntf[:, 0:1].reshape(1, N, 1)


def _edge_scatter_jnp(e3, dstp, ne):
    acc = jax.ops.segment_sum(e3[:ne], dstp[:ne], num_segments=N)
    cnt = jax.ops.segment_sum(jnp.ones((ne,), jnp.float32), dstp[:ne],
                              num_segments=N)
    return acc.reshape(1, N, H), cnt.reshape(1, N, 1)


def _edge_mlp(xi, xj, q, ne):
    e1, st1 = _mlp_pass1(xi, xj, q["m1"]["W"].T, q["m1"]["b"], ne)
    e2, st2 = _mlp_pass(e1, st1, q["m2"]["W"].T, q["m2"]["b"],
                        q["m1"]["g"], q["m1"]["be"], ne)
    e3, st3 = _mlp_pass(e2, st2, q["m3"]["W"].T, q["m3"]["b"],
                        q["m2"]["g"], q["m2"]["be"], ne)
    return e3, st3


def _edge_conv(xin, src, dst, q):
    sc_pre = _node_mm(xin, q["sc"]["W"].T, q["sc"]["b"])
    ep = _epad(E)
    pad = ep - E
    zpad = jnp.zeros((pad,), jnp.int32)
    srcp = jnp.concatenate([src, zpad])
    dstg = jnp.concatenate([dst, zpad])
    dsts = jnp.concatenate([dst, jnp.full((pad,), _DUMP, jnp.int32)])
    xi, xj = _edge_gather(xin, jnp.stack([dstg, srcp]), ep)
    e3, st3 = _edge_mlp(xi, xj, q, E)
    acc, cnt = _edge_scatter_jnp(e3, dsts, E)
    return _econv_finalize(acc, cnt, st3, q["m3"], sc_pre, q["sc"], E)


def _knn_conv(xin, br, bc, dstk, q):
    sc_pre = _node_mm(xin, q["sc"]["W"].T, q["sc"]["b"])
    d2 = jnp.sum(xin * xin, axis=1)
    x_pad = jnp.concatenate(
        [xin, jnp.zeros((_NP_TK - N, D), jnp.float32)], axis=0)
    d2r_pad = jnp.concatenate(
        [d2, jnp.zeros((_NP_TK - N,), jnp.float32)]).reshape(1, _NP_TK)
    bc_pad = jnp.concatenate(
        [bc[0], jnp.full((_NP_TK - N,), -1, jnp.int32)]).reshape(1, _NP_TK)
    idx = _knn_topk(x_pad, d2.reshape(N, 1), d2r_pad, br, bc_pad)
    nk = N * K
    ep = _epad(nk)
    zpad = jnp.zeros((ep - nk,), jnp.int32)
    srck = jnp.concatenate([idx.reshape(-1), zpad])
    xj = _gather1(xin, srck.reshape(1, ep), ep)
    e1, st1 = _mlp_pass1k(x_pad, xj, q["m1"]["W"].T, q["m1"]["b"], nk)
    e2, st2 = _mlp_pass(e1, st1, q["m2"]["W"].T, q["m2"]["b"],
                        q["m1"]["g"], q["m1"]["be"], nk)
    e3, st3 = _mlp_pass(e2, st2, q["m3"]["W"].T, q["m3"]["b"],
                        q["m2"]["g"], q["m2"]["be"], nk)
    return _knn_finalize(e3[:nk].reshape(N, K * H), st3, q["m3"],
                         sc_pre, q["sc"], nk)


def kernel(x, edge_index, graph_input, batch, params):
    br = batch.reshape(N, 1)
    bc = batch.reshape(1, N)
    src = edge_index[0]
    dst = edge_index[1]
    dstk = jnp.repeat(jnp.arange(N, dtype=jnp.int32), K)

    bc3 = batch.reshape(N // _RB_N, 1, _RB_N)
    xg = _graph_norm(x, br, bc3, params["gn"])
    c1 = _edge_conv(xg, src, dst, params["conv1"])
    c2 = _knn_conv(c1, br, bc, dstk, params["conv2"])
    c3 = _knn_conv(c2, br, bc, dstk, params["conv3"])
    return _head(c1, c2, c3, graph_input, bc3, params["head"])


# speedup vs baseline: 1.3015x; 1.0006x over previous
import functools

import jax
import jax.numpy as jnp
from jax import lax
from jax.experimental import pallas as pl
from jax.experimental.pallas import tpu as pltpu

N = 10000
E = 320000
D = 128
H = 128
G = 256
GF = 4
NC = 2
K = 4
EPS = 1e-5

_HI = lax.Precision.HIGHEST


def _lrelu(x):
    return jnp.where(x >= 0, x, 0.01 * x)


def _pc(body, out_shape, grid=None, in_specs=None, out_specs=None, scratch_shapes=None):
    kw = {}
    if grid is not None:
        kw["grid"] = grid
    if in_specs is not None:
        kw["in_specs"] = in_specs
    if out_specs is not None:
        kw["out_specs"] = out_specs
    if scratch_shapes is not None:
        kw["scratch_shapes"] = scratch_shapes
    return pl.pallas_call(body, out_shape=out_shape, **kw)


_RB_N = 2000


def _segsum_body(y_ref, bc_ref, s_ref, c_ref, accs, accc):
    i = pl.program_id(0)
    iota_col = lax.broadcasted_iota(jnp.int32, (G, 1), 0)
    MT = (iota_col == bc_ref[0]).astype(jnp.float32)
    ps = jnp.dot(MT, y_ref[...], precision=_HI, preferred_element_type=jnp.float32)
    pc_ = jnp.sum(MT, axis=1, keepdims=True)

    @pl.when(i == 0)
    def _():
        accs[...] = jnp.zeros_like(accs)
        accc[...] = jnp.zeros_like(accc)

    accs[...] += ps
    accc[...] += pc_

    @pl.when(i == pl.num_programs(0) - 1)
    def _():
        s_ref[...] = accs[...]
        c_ref[...] = accc[...]


def _segsum(y, bc3):
    c = y.shape[1]
    return _pc(
        _segsum_body,
        (jax.ShapeDtypeStruct((G, c), jnp.float32),
         jax.ShapeDtypeStruct((G, 1), jnp.float32)),
        grid=(N // _RB_N,),
        in_specs=[pl.BlockSpec((_RB_N, c), lambda i: (i, 0)),
                  pl.BlockSpec((1, 1, _RB_N), lambda i: (i, 0, 0))],
        out_specs=(pl.BlockSpec((G, c), lambda i: (0, 0)),
                   pl.BlockSpec((G, 1), lambda i: (0, 0))),
        scratch_shapes=[pltpu.VMEM((G, c), jnp.float32),
                        pltpu.VMEM((G, 1), jnp.float32)],
    )(y, bc3)


def _gn_mid_body(x_ref, br_ref, bc_ref, s_ref, c_ref, ms_ref,
                 o_ref, v_ref, accv):
    i = pl.program_id(0)
    cnt = jnp.maximum(c_ref[...], 1.0)
    mean = s_ref[...] / cnt
    iota_row = lax.broadcasted_iota(jnp.int32, (1, G), 1)
    M = (br_ref[...] == iota_row).astype(jnp.float32)
    meanb = jnp.dot(M, mean, precision=_HI, preferred_element_type=jnp.float32)
    out0 = x_ref[...] - meanb * ms_ref[...]
    o_ref[...] = out0
    iota_col = lax.broadcasted_iota(jnp.int32, (G, 1), 0)
    MT = (iota_col == bc_ref[0]).astype(jnp.float32)
    pv = jnp.dot(MT, out0 * out0, precision=_HI,
                 preferred_element_type=jnp.float32)

    @pl.when(i == 0)
    def _():
        accv[...] = jnp.zeros_like(accv)

    accv[...] += pv

    @pl.when(i == pl.num_programs(0) - 1)
    def _():
        v_ref[...] = accv[...]


def _gn_fin_body(o0_ref, br_ref, v_ref, c_ref, w_ref, b_ref, o_ref):
    cnt = jnp.maximum(c_ref[...], 1.0)
    var = v_ref[...] / cnt
    iota_row = lax.broadcasted_iota(jnp.int32, (1, G), 1)
    M = (br_ref[...] == iota_row).astype(jnp.float32)
    varb = jnp.dot(M, var, precision=_HI, preferred_element_type=jnp.float32)
    o_ref[...] = w_ref[...] * o0_ref[...] / jnp.sqrt(varb + EPS) + b_ref[...]


def _graph_norm(x, br, bc3, p):
    sums, cnt = _segsum(x, bc3)
    out0, vsums = _pc(
        _gn_mid_body,
        (jax.ShapeDtypeStruct((N, D), jnp.float32),
         jax.ShapeDtypeStruct((G, D), jnp.float32)),
        grid=(N // _RB_N,),
        in_specs=[pl.BlockSpec((_RB_N, D), lambda i: (i, 0)),
                  pl.BlockSpec((_RB_N, 1), lambda i: (i, 0)),
                  pl.BlockSpec((1, 1, _RB_N), lambda i: (i, 0, 0)),
                  pl.BlockSpec((G, D), lambda i: (0, 0)),
                  pl.BlockSpec((G, 1), lambda i: (0, 0)),
                  pl.BlockSpec((1, D), lambda i: (0, 0))],
        out_specs=(pl.BlockSpec((_RB_N, D), lambda i: (i, 0)),
                   pl.BlockSpec((G, D), lambda i: (0, 0))),
        scratch_shapes=[pltpu.VMEM((G, D), jnp.float32)],
    )(x, br, bc3, sums, cnt, p["ms"].reshape(1, D))
    return _pc(
        _gn_fin_body,
        jax.ShapeDtypeStruct((N, D), jnp.float32),
        grid=(N // _RB_N,),
        in_specs=[pl.BlockSpec((_RB_N, D), lambda i: (i, 0)),
                  pl.BlockSpec((_RB_N, 1), lambda i: (i, 0)),
                  pl.BlockSpec((G, D), lambda i: (0, 0)),
                  pl.BlockSpec((G, 1), lambda i: (0, 0)),
                  pl.BlockSpec((1, D), lambda i: (0, 0)),
                  pl.BlockSpec((1, D), lambda i: (0, 0))],
        out_specs=pl.BlockSpec((_RB_N, D), lambda i: (i, 0)),
    )(out0, br, vsums, cnt, p["w"].reshape(1, D), p["b"].reshape(1, D))


def _mm_body(x_ref, w_ref, b_ref, o_ref):
    o_ref[...] = jnp.dot(x_ref[...], w_ref[...],
                         preferred_element_type=jnp.float32) + b_ref[...]


def _node_mm(x, wt, b):
    n, _ = x.shape
    co = wt.shape[1]
    return _pc(_mm_body, jax.ShapeDtypeStruct((n, co), jnp.float32))(
        x, wt, b.reshape(1, co))


_RB_E = 4096


def _mlp1_body(ne, xi_ref, xj_ref, w_ref, b_ref, o_ref, so_ref, acc):
    i = pl.program_id(0)
    xi = xi_ref[...]
    cat = jnp.concatenate([xi, xj_ref[...] - xi], axis=1)
    out = _lrelu(jnp.dot(cat, w_ref[...],
                         preferred_element_type=jnp.float32) + b_ref[...])
    o_ref[...] = out

    @pl.when(i == 0)
    def _():
        acc[...] = jnp.zeros_like(acc)

    rb = out.shape[0]
    gid = i * rb + lax.broadcasted_iota(jnp.int32, (rb, 1), 0)
    outm = jnp.where(gid < ne, out, 0.0)
    acc[0:1, :] += jnp.sum(outm, axis=0, keepdims=True)
    acc[1:2, :] += jnp.sum(outm * outm, axis=0, keepdims=True)

    @pl.when(i == pl.num_programs(0) - 1)
    def _():
        so_ref[...] = acc[...]


def _mlp_pass1(xi, xj, wt, b, ne):
    ep = xi.shape[0]
    nblk = ep // _RB_E
    body = functools.partial(_mlp1_body, ne)
    return _pc(
        body,
        (jax.ShapeDtypeStruct((ep, H), jnp.float32),
         jax.ShapeDtypeStruct((2, H), jnp.float32)),
        grid=(nblk,),
        in_specs=[
            pl.BlockSpec((_RB_E, H), lambda i: (i, 0)),
            pl.BlockSpec((_RB_E, H), lambda i: (i, 0)),
            pl.BlockSpec((2 * H, H), lambda i: (0, 0)),
            pl.BlockSpec((1, H), lambda i: (0, 0)),
        ],
        out_specs=(pl.BlockSpec((_RB_E, H), lambda i: (i, 0)),
                   pl.BlockSpec((2, H), lambda i: (0, 0))),
        scratch_shapes=[pltpu.VMEM((2, H), jnp.float32)],
    )(xi, xj, wt, b.reshape(1, H))


def _mlp1k_body(ne, xn_ref, xj_ref, w_ref, b_ref, o_ref, so_ref, acc):
    i = pl.program_id(0)
    xn = xn_ref[...]
    rbn = xn.shape[0]
    xi = jnp.reshape(jnp.broadcast_to(xn[:, None, :], (rbn, K, H)),
                     (rbn * K, H))
    cat = jnp.concatenate([xi, xj_ref[...] - xi], axis=1)
    out = _lrelu(jnp.dot(cat, w_ref[...],
                         preferred_element_type=jnp.float32) + b_ref[...])
    o_ref[...] = out

    @pl.when(i == 0)
    def _():
        acc[...] = jnp.zeros_like(acc)

    rb = out.shape[0]
    gid = i * rb + lax.broadcasted_iota(jnp.int32, (rb, 1), 0)
    outm = jnp.where(gid < ne, out, 0.0)
    acc[0:1, :] += jnp.sum(outm, axis=0, keepdims=True)
    acc[1:2, :] += jnp.sum(outm * outm, axis=0, keepdims=True)

    @pl.when(i == pl.num_programs(0) - 1)
    def _():
        so_ref[...] = acc[...]


def _mlp_pass1k(x_pad, xj, wt, b, ne):
    ep = xj.shape[0]
    nblk = ep // _RB_E
    body = functools.partial(_mlp1k_body, ne)
    return _pc(
        body,
        (jax.ShapeDtypeStruct((ep, H), jnp.float32),
         jax.ShapeDtypeStruct((2, H), jnp.float32)),
        grid=(nblk,),
        in_specs=[
            pl.BlockSpec((_RB_E // K, H), lambda i: (i, 0)),
            pl.BlockSpec((_RB_E, H), lambda i: (i, 0)),
            pl.BlockSpec((2 * H, H), lambda i: (0, 0)),
            pl.BlockSpec((1, H), lambda i: (0, 0)),
        ],
        out_specs=(pl.BlockSpec((_RB_E, H), lambda i: (i, 0)),
                   pl.BlockSpec((2, H), lambda i: (0, 0))),
        scratch_shapes=[pltpu.VMEM((2, H), jnp.float32)],
    )(x_pad, xj, wt, b.reshape(1, H))


def _mlp_pass_body(ne, e_ref, st_ref, w_ref, bcur_ref, gprev_ref, beprev_ref,
                   o_ref, so_ref, acc):
    i = pl.program_id(0)
    st = st_ref[...]
    mean = st[0:1, :] / ne
    var = st[1:2, :] / ne - mean * mean
    en = (e_ref[...] - mean) / jnp.sqrt(var + EPS) * gprev_ref[...] + beprev_ref[...]
    out = _lrelu(jnp.dot(en, w_ref[...], preferred_element_type=jnp.float32)
                 + bcur_ref[...])
    o_ref[...] = out

    @pl.when(i == 0)
    def _():
        acc[...] = jnp.zeros_like(acc)

    rb = out.shape[0]
    gid = i * rb + lax.broadcasted_iota(jnp.int32, (rb, 1), 0)
    outm = jnp.where(gid < ne, out, 0.0)
    acc[0:1, :] += jnp.sum(outm, axis=0, keepdims=True)
    acc[1:2, :] += jnp.sum(outm * outm, axis=0, keepdims=True)

    @pl.when(i == pl.num_programs(0) - 1)
    def _():
        so_ref[...] = acc[...]


def _mlp_pass(e, stats, wt, b_cur, g_prev, be_prev, ne):
    ep = e.shape[0]
    nblk = ep // _RB_E
    body = functools.partial(_mlp_pass_body, float(ne))
    return _pc(
        body,
        (jax.ShapeDtypeStruct((ep, H), jnp.float32),
         jax.ShapeDtypeStruct((2, H), jnp.float32)),
        grid=(nblk,),
        in_specs=[
            pl.BlockSpec((_RB_E, H), lambda i: (i, 0)),
            pl.BlockSpec((2, H), lambda i: (0, 0)),
            pl.BlockSpec((H, H), lambda i: (0, 0)),
            pl.BlockSpec((1, H), lambda i: (0, 0)),
            pl.BlockSpec((1, H), lambda i: (0, 0)),
            pl.BlockSpec((1, H), lambda i: (0, 0)),
        ],
        out_specs=(pl.BlockSpec((_RB_E, H), lambda i: (i, 0)),
                   pl.BlockSpec((2, H), lambda i: (0, 0))),
        scratch_shapes=[pltpu.VMEM((2, H), jnp.float32)],
    )(e, stats, wt, b_cur.reshape(1, H), g_prev.reshape(1, H), be_prev.reshape(1, H))


def _bn_sc(sc, g, be):
    m = jnp.mean(sc, axis=0, keepdims=True)
    d = sc - m
    v = jnp.mean(d * d, axis=0, keepdims=True)
    return d / jnp.sqrt(v + EPS) * g + be


def _econv_fin_body(ne, acc_ref, cnt_ref, st3_ref, g3_ref, be3_ref,
                    sc_ref, gsc_ref, besc_ref, o_ref):
    st = st3_ref[...]
    mean3 = st[0:1, :] / ne
    var3 = st[1:2, :] / ne - mean3 * mean3
    s3 = g3_ref[...] / jnp.sqrt(var3 + EPS)
    c3 = be3_ref[...] - mean3 * s3
    accs = jnp.sum(acc_ref[...], axis=0)
    cnt = jnp.sum(cnt_ref[...], axis=0)
    agg = (accs * s3 + c3 * cnt) / jnp.maximum(cnt, 1.0)
    o_ref[...] = agg + _bn_sc(sc_ref[...], gsc_ref[...], besc_ref[...])


def _econv_finalize(acc, cnt, st3, q3, sc_pre, qsc, ne):
    body = functools.partial(_econv_fin_body, float(ne))
    return _pc(body, jax.ShapeDtypeStruct((N, H), jnp.float32))(
        acc, cnt, st3, q3["g"].reshape(1, H), q3["be"].reshape(1, H),
        sc_pre, qsc["g"].reshape(1, H), qsc["be"].reshape(1, H))


def _knn_fin_body(ne, e_ref, st3_ref, g3_ref, be3_ref, sc_ref, gsc_ref,
                  besc_ref, o_ref):
    st = st3_ref[...]
    mean3 = st[0:1, :] / ne
    var3 = st[1:2, :] / ne - mean3 * mean3
    s3 = g3_ref[...] / jnp.sqrt(var3 + EPS)
    c3 = be3_ref[...] - mean3 * s3
    e = e_ref[...]
    mn = (e[:, 0:H] + e[:, H:2 * H] + e[:, 2 * H:3 * H] + e[:, 3 * H:4 * H]) * 0.25
    agg = mn * s3 + c3
    o_ref[...] = agg + _bn_sc(sc_ref[...], gsc_ref[...], besc_ref[...])


def _knn_finalize(e3r, st3, q3, sc_pre, qsc, ne):
    body = functools.partial(_knn_fin_body, float(ne))
    return _pc(body, jax.ShapeDtypeStruct((N, H), jnp.float32))(
        e3r, st3, q3["g"].reshape(1, H), q3["be"].reshape(1, H),
        sc_pre, qsc["g"].reshape(1, H), qsc["be"].reshape(1, H))


_RB_TK = 400
_CT_TK = 512
_NP_TK = 10240
_BIGI = 1 << 30


def _topk_body(x_ref, d2c_ref, d2r_ref, br_blk_ref, bc_ref, idx_ref):
    i = pl.program_id(0)
    R = _RB_TK
    CT = _CT_TK
    xr = x_ref[pl.ds(i * R, R), :]
    d2r_blk = d2c_ref[pl.ds(i * R, R), :]
    br = br_blk_ref[...]
    b_lo = br[0, 0]
    b_hi = br[R - 1, 0]
    iota_n = lax.broadcasted_iota(jnp.int32, (1, _NP_TK), 1)
    bc_all = bc_ref[...]
    cmin = jnp.min(jnp.where(bc_all[0:1, :] == b_lo, iota_n, _BIGI))
    cmax = jnp.max(jnp.where(bc_all[0:1, :] == b_hi, iota_n, -1)) + 1
    t0 = cmin // CT
    t1 = (cmax + CT - 1) // CT
    row_ids = i * R + lax.broadcasted_iota(jnp.int32, (R, 1), 0)
    inf = jnp.float32(jnp.inf)

    def _sel4(nv, ni):
        bvs = []
        bis = []
        for _ in range(K):
            m = jnp.min(nv, axis=1, keepdims=True)
            mi = jnp.min(jnp.where(nv == m, ni, _BIGI), axis=1, keepdims=True)
            nv = jnp.where(ni == mi, inf, nv)
            bvs.append(m)
            bis.append(mi)
        return jnp.concatenate(bvs, axis=1), jnp.concatenate(bis, axis=1)

    def tile_step(t, carry):
        bv, bi = carry
        c0 = pl.multiple_of(t * CT, CT)
        xc = x_ref[pl.ds(c0, CT), :]
        qk = lax.dot_general(xr, xc, (((1,), (1,)), ((), ())),
                             preferred_element_type=jnp.float32)
        d2c_row = d2r_ref[:, pl.ds(c0, CT)]
        dist = d2r_blk + d2c_row - 2.0 * qk
        bct = bc_ref[:, pl.ds(c0, CT)]
        col_ids = c0 + lax.broadcasted_iota(jnp.int32, (1, CT), 1)
        bad = (bct != br) | (col_ids == row_ids)
        cand = jnp.where(bad, inf, dist)
        tv = []
        ti = []
        for _ in range(K):
            tm = jnp.min(cand, axis=1, keepdims=True)
            tix = jnp.min(jnp.where(cand == tm, col_ids, _BIGI),
                          axis=1, keepdims=True)
            cand = jnp.where(col_ids == tix, inf, cand)
            tv.append(tm)
            ti.append(tix)
        nv = jnp.concatenate([bv] + tv, axis=1)
        ni = jnp.concatenate([bi] + ti, axis=1)
        return _sel4(nv, ni)

    bv0 = jnp.full((R, K), inf, jnp.float32)
    bi0 = jnp.full((R, K), _BIGI, jnp.int32)
    bv, bi = lax.fori_loop(t0, t1, tile_step, (bv0, bi0))

    s0 = t0 * CT
    s1 = jnp.minimum(t1 * CT, N)
    kk = lax.broadcasted_iota(jnp.int32, (1, K), 1)
    ids_out = jnp.where(kk < s0, kk, s1 + kk - s0)
    nv = jnp.concatenate([bv, jnp.full((R, K), inf, jnp.float32)], axis=1)
    ni = jnp.concatenate([bi, jnp.broadcast_to(ids_out, (R, K))], axis=1)
    bv, bi = _sel4(nv, ni)
    idx_ref[...] = bi


def _knn_topk(x_pad, d2c, d2r_pad, br, bc_pad):
    return _pc(
        _topk_body,
        jax.ShapeDtypeStruct((N, K), jnp.int32),
        grid=(N // _RB_TK,),
        in_specs=[
            pl.BlockSpec((_NP_TK, D), lambda i: (0, 0)),
            pl.BlockSpec((N, 1), lambda i: (0, 0)),
            pl.BlockSpec((1, _NP_TK), lambda i: (0, 0)),
            pl.BlockSpec((_RB_TK, 1), lambda i: (i, 0)),
            pl.BlockSpec((1, _NP_TK), lambda i: (0, 0)),
        ],
        out_specs=pl.BlockSpec((_RB_TK, K), lambda i: (i, 0)),
    )(x_pad, d2c, d2r_pad, br, bc_pad)


def _bn_rows(x, g, be):
    m = jnp.mean(x, axis=0, keepdims=True)
    d = x - m
    v = jnp.mean(d * d, axis=0, keepdims=True)
    return d / jnp.sqrt(v + EPS) * g + be


def _head_body(p1_ref, p2_ref, p3_ref, gi_ref, cnt_ref,
               g0a_ref, b0a_ref, g0b_ref, b0b_ref, g0c_ref, b0c_ref,
               g0g_ref, b0g_ref,
               w1a_ref, w1b_ref, w1c_ref, w1g_ref, b1_ref, g1_ref, be1_ref,
               w2_ref, b2_ref, g2_ref, be2_ref, wo_ref, bo_ref, o_ref):
    cnt = jnp.maximum(cnt_ref[...], 1.0)
    p1 = p1_ref[...] / cnt
    p2 = p2_ref[...] / cnt
    p3 = p3_ref[...] / cnt
    gi = gi_ref[...]
    p1 = _bn_rows(p1, g0a_ref[...], b0a_ref[...])
    p2 = _bn_rows(p2, g0b_ref[...], b0b_ref[...])
    p3 = _bn_rows(p3, g0c_ref[...], b0c_ref[...])
    gi = _bn_rows(gi, g0g_ref[...], b0g_ref[...])
    h = (jnp.dot(p1, w1a_ref[...], preferred_element_type=jnp.float32)
         + jnp.dot(p2, w1b_ref[...], preferred_element_type=jnp.float32)
         + jnp.dot(p3, w1c_ref[...], preferred_element_type=jnp.float32)
         + jnp.dot(gi, w1g_ref[...], preferred_element_type=jnp.float32)
         + b1_ref[...])
    h = _bn_rows(_lrelu(h), g1_ref[...], be1_ref[...])
    h = _lrelu(jnp.dot(h, w2_ref[...], preferred_element_type=jnp.float32)
               + b2_ref[...])
    h = _bn_rows(h, g2_ref[...], be2_ref[...])
    lo = jnp.dot(h, wo_ref[...], preferred_element_type=jnp.float32) + bo_ref[...]
    m = jnp.max(lo, axis=1, keepdims=True)
    ex = jnp.exp(lo - m)
    o_ref[...] = ex / jnp.sum(ex, axis=1, keepdims=True)


def _head(c1, c2, c3, gi, bc3, hp):
    p1s, cnt = _segsum(c1, bc3)
    p2s, _ = _segsum(c2, bc3)
    p3s, _ = _segsum(c3, bc3)
    w1t = hp["d1_W"].T
    args = [
        p1s, p2s, p3s, gi, cnt,
        hp["bn0_g"][0:H].reshape(1, H), hp["bn0_b"][0:H].reshape(1, H),
        hp["bn0_g"][H:2 * H].reshape(1, H), hp["bn0_b"][H:2 * H].reshape(1, H),
        hp["bn0_g"][2 * H:3 * H].reshape(1, H), hp["bn0_b"][2 * H:3 * H].reshape(1, H),
        hp["bn0_g"][3 * H:].reshape(1, GF), hp["bn0_b"][3 * H:].reshape(1, GF),
        w1t[0:H], w1t[H:2 * H], w1t[2 * H:3 * H], w1t[3 * H:],
        hp["d1_b"].reshape(1, H), hp["bn1_g"].reshape(1, H), hp["bn1_b"].reshape(1, H),
        hp["d2_W"].T, hp["d2_b"].reshape(1, H),
        hp["bn2_g"].reshape(1, H), hp["bn2_b"].reshape(1, H),
        hp["out_W"].T, hp["out_b"].reshape(1, NC),
    ]
    return _pc(_head_body, jax.ShapeDtypeStruct((G, NC), jnp.float32))(*args)


_CE = 128
_NW = 32


def _epad(ne):
    return ((ne + _NW * _CE - 1) // (_NW * _CE)) * (_NW * _CE)


def _edge_gather(x, sd2, ep):
    from jax.experimental.pallas import tpu_sc as plsc
    nch = ep // (_NW * _CE)
    mesh = plsc.VectorSubcoreMesh(core_axis_name="c", subcore_axis_name="s")

    @functools.partial(
        pl.kernel, mesh=mesh,
        out_type=[jax.ShapeDtypeStruct((ep, H), jnp.float32),
                  jax.ShapeDtypeStruct((ep, H), jnp.float32)],
        scratch_types=[
            pltpu.VMEM((2, _CE), jnp.int32),
            pltpu.VMEM((2, _CE), jnp.int32),
            pltpu.VMEM((_CE, H), jnp.float32),
            pltpu.VMEM((_CE, H), jnp.float32),
            pltpu.VMEM((_CE, H), jnp.float32),
            pltpu.VMEM((_CE, H), jnp.float32),
            pltpu.SemaphoreType.DMA,
            pltpu.SemaphoreType.DMA,
            pltpu.SemaphoreType.DMA,
            pltpu.SemaphoreType.DMA,
        ])
    def k(x_hbm, sd_hbm, oi_hbm, oj_hbm,
          ix0, ix1, bi0, bj0, bi1, bj1, sg0, sg1, so0, so1):
        wid = lax.axis_index("s") * 2 + lax.axis_index("c")
        base0 = wid * (ep // _NW)
        ix = (ix0, ix1)
        bi = (bi0, bi1)
        bj = (bj0, bj1)
        sg = (sg0, sg1)
        so = (so0, so1)

        def load_fire(j, s):
            base = base0 + j * _CE
            pltpu.sync_copy(sd_hbm.at[:, pl.ds(base, _CE)], ix[s])
            pltpu.async_copy(x_hbm.at[ix[s].at[0]], bi[s], sg[s])
            pltpu.async_copy(x_hbm.at[ix[s].at[1]], bj[s], sg[s])

        def drain_g(s):
            pltpu.make_async_copy(x_hbm.at[ix[s].at[0]], bi[s], sg[s]).wait()
            pltpu.make_async_copy(x_hbm.at[ix[s].at[1]], bj[s], sg[s]).wait()

        def fire_out(j, s):
            base = base0 + j * _CE
            pltpu.async_copy(bi[s], oi_hbm.at[pl.ds(base, _CE)], so[s])
            pltpu.async_copy(bj[s], oj_hbm.at[pl.ds(base, _CE)], so[s])

        def drain_out(s):
            pltpu.make_async_copy(bi[s], oi_hbm.at[pl.ds(0, _CE)], so[s]).wait()
            pltpu.make_async_copy(bj[s], oj_hbm.at[pl.ds(0, _CE)], so[s]).wait()

        load_fire(0, 0)

        def pair(t2, carry):
            for s in (0, 1):
                j = 2 * t2 + s

                @pl.when(j < nch)
                def _():
                    drain_g(s)

                    @pl.when(j + 1 < nch)
                    def _():
                        @pl.when(j >= 1)
                        def _():
                            drain_out(1 - s)

                        load_fire(j + 1, 1 - s)

                    fire_out(j, s)
            return carry

        lax.fori_loop(0, (nch + 1) // 2, pair, 0)
        drain_out((nch - 1) % 2)
        drain_out(nch % 2)

    return k(x, sd2)


def _gather1(x, idxp, ep):
    from jax.experimental.pallas import tpu_sc as plsc
    nch = ep // (_NW * _CE)
    mesh = plsc.VectorSubcoreMesh(core_axis_name="c", subcore_axis_name="s")

    @functools.partial(
        pl.kernel, mesh=mesh,
        out_type=jax.ShapeDtypeStruct((ep, H), jnp.float32),
        scratch_types=[
            pltpu.VMEM((1, _CE), jnp.int32),
            pltpu.VMEM((1, _CE), jnp.int32),
            pltpu.VMEM((_CE, H), jnp.float32),
            pltpu.VMEM((_CE, H), jnp.float32),
            pltpu.SemaphoreType.DMA,
            pltpu.SemaphoreType.DMA,
            pltpu.SemaphoreType.DMA,
            pltpu.SemaphoreType.DMA,
        ])
    def k(x_hbm, idx_hbm, o_hbm, ix0, ix1, b0, b1, sg0, sg1, so0, so1):
        wid = lax.axis_index("s") * 2 + lax.axis_index("c")
        base0 = wid * (ep // _NW)
        ix = (ix0, ix1)
        bb = (b0, b1)
        sg = (sg0, sg1)
        so = (so0, so1)

        def load_fire(j, s):
            base = base0 + j * _CE
            pltpu.sync_copy(idx_hbm.at[:, pl.ds(base, _CE)], ix[s])
            pltpu.async_copy(x_hbm.at[ix[s].at[0]], bb[s], sg[s])

        def drain_g(s):
            pltpu.make_async_copy(x_hbm.at[ix[s].at[0]], bb[s], sg[s]).wait()

        def fire_out(j, s):
            base = base0 + j * _CE
            pltpu.async_copy(bb[s], o_hbm.at[pl.ds(base, _CE)], so[s])

        def drain_out(s):
            pltpu.make_async_copy(bb[s], o_hbm.at[pl.ds(0, _CE)], so[s]).wait()

        load_fire(0, 0)

        def pair(t2, carry):
            for s in (0, 1):
                j = 2 * t2 + s

                @pl.when(j < nch)
                def _():
                    drain_g(s)

                    @pl.when(j + 1 < nch)
                    def _():
                        @pl.when(j >= 1)
                        def _():
                            drain_out(1 - s)

                        load_fire(j + 1, 1 - s)

                    fire_out(j, s)
            return carry

        lax.fori_loop(0, (nch + 1) // 2, pair, 0)
        drain_out((nch - 1) % 2)
        drain_out(nch % 2)

    return k(x, idxp)


def _edge_scatter_jnp(e3, dstp, ne):
    acc = jax.ops.segment_sum(e3[:ne], dstp[:ne], num_segments=N)
    cnt = jax.ops.segment_sum(jnp.ones((ne,), jnp.float32), dstp[:ne],
                              num_segments=N)
    return acc.reshape(1, N, H), cnt.reshape(1, N, 1)


def _edge_mlp(xi, xj, q, ne):
    e1, st1 = _mlp_pass1(xi, xj, q["m1"]["W"].T, q["m1"]["b"], ne)
    e2, st2 = _mlp_pass(e1, st1, q["m2"]["W"].T, q["m2"]["b"],
                        q["m1"]["g"], q["m1"]["be"], ne)
    e3, st3 = _mlp_pass(e2, st2, q["m3"]["W"].T, q["m3"]["b"],
                        q["m2"]["g"], q["m2"]["be"], ne)
    return e3, st3


def _edge_conv(xin, src, dst, q):
    sc_pre = _node_mm(xin, q["sc"]["W"].T, q["sc"]["b"])
    ep = _epad(E)
    pad = ep - E
    zpad = jnp.zeros((pad,), jnp.int32)
    srcp = jnp.concatenate([src, zpad])
    dstg = jnp.concatenate([dst, zpad])
    xi, xj = _edge_gather(xin, jnp.stack([dstg, srcp]), ep)
    e3, st3 = _edge_mlp(xi, xj, q, E)
    acc, cnt = _edge_scatter_jnp(e3, dst, E)
    return _econv_finalize(acc, cnt, st3, q["m3"], sc_pre, q["sc"], E)


def _knn_conv(xin, br, bc, q):
    sc_pre = _node_mm(xin, q["sc"]["W"].T, q["sc"]["b"])
    d2 = jnp.sum(xin * xin, axis=1)
    x_pad = jnp.concatenate(
        [xin, jnp.zeros((_NP_TK - N, D), jnp.float32)], axis=0)
    d2r_pad = jnp.concatenate(
        [d2, jnp.zeros((_NP_TK - N,), jnp.float32)]).reshape(1, _NP_TK)
    bc_pad = jnp.concatenate(
        [bc[0], jnp.full((_NP_TK - N,), -1, jnp.int32)]).reshape(1, _NP_TK)
    idx = _knn_topk(x_pad, d2.reshape(N, 1), d2r_pad, br, bc_pad)
    nk = N * K
    ep = _epad(nk)
    zpad = jnp.zeros((ep - nk,), jnp.int32)
    srck = jnp.concatenate([idx.reshape(-1), zpad])
    xj = _gather1(xin, srck.reshape(1, ep), ep)
    e1, st1 = _mlp_pass1k(x_pad, xj, q["m1"]["W"].T, q["m1"]["b"], nk)
    e2, st2 = _mlp_pass(e1, st1, q["m2"]["W"].T, q["m2"]["b"],
                        q["m1"]["g"], q["m1"]["be"], nk)
    e3, st3 = _mlp_pass(e2, st2, q["m3"]["W"].T, q["m3"]["b"],
                        q["m2"]["g"], q["m2"]["be"], nk)
    return _knn_finalize(e3[:nk].reshape(N, K * H), st3, q["m3"],
                         sc_pre, q["sc"], nk)


def kernel(x, edge_index, graph_input, batch, params):
    br = batch.reshape(N, 1)
    bc = batch.reshape(1, N)
    src = edge_index[0]
    dst = edge_index[1]
    bc3 = batch.reshape(N // _RB_N, 1, _RB_N)
    xg = _graph_norm(x, br, bc3, params["gn"])
    c1 = _edge_conv(xg, src, dst, params["conv1"])
    c2 = _knn_conv(c1, br, bc, params["conv2"])
    c3 = _knn_conv(c2, br, bc, params["conv3"])
    return _head(c1, c2, c3, graph_input, bc3, params["head"])
